# baseline (speedup 1.0000x reference)
import functools
import jax
import jax.numpy as jnp
from jax.experimental import pallas as pl

N_NODES = 10000
N_EDGES = 320000
D = 128
BLK = 2560
assert N_EDGES % BLK == 0


def _t2_body(m_ref, w_ref, r_ref, e0_ref, y_ref):
    acc = jnp.dot(m_ref[...], w_ref[...], preferred_element_type=jnp.float32,
                  precision=jax.lax.Precision.HIGHEST)
    y_ref[...] = jnp.maximum(acc + r_ref[...] + e0_ref[...], 0.0)


def _t2(m, wbp_t, r, e0):
    grid = (N_EDGES // BLK,)
    return pl.pallas_call(
        _t2_body,
        grid=grid,
        in_specs=[
            pl.BlockSpec((BLK, D), lambda i: (i, 0)),
            pl.BlockSpec((D, D), lambda i: (0, 0)),
            pl.BlockSpec((BLK, D), lambda i: (i, 0)),
            pl.BlockSpec((1, D), lambda i: (0, 0)),
        ],
        out_specs=pl.BlockSpec((BLK, D), lambda i: (i, 0)),
        out_shape=jax.ShapeDtypeStruct((N_EDGES, D), jnp.float32),
    )(m, wbp_t, r, e0)


def _tg_body(m_ref, g_ref, s_ref):
    @pl.when(pl.program_id(0) == 0)
    def _():
        g_ref[...] = jnp.zeros_like(g_ref)
        s_ref[...] = jnp.zeros_like(s_ref)

    blk = m_ref[...]
    g_ref[...] += jnp.dot(blk.T, blk, preferred_element_type=jnp.float32,
                          precision=jax.lax.Precision.HIGHEST)
    s_ref[...] += jnp.sum(blk, axis=0, keepdims=True)


def _tg(m):
    grid = (N_EDGES // BLK,)
    return pl.pallas_call(
        _tg_body,
        grid=grid,
        in_specs=[pl.BlockSpec((BLK, D), lambda i: (i, 0))],
        out_specs=[
            pl.BlockSpec((D, D), lambda i: (0, 0)),
            pl.BlockSpec((1, D), lambda i: (0, 0)),
        ],
        out_shape=[
            jax.ShapeDtypeStruct((D, D), jnp.float32),
            jax.ShapeDtypeStruct((1, D), jnp.float32),
        ],
    )(m)


def kernel(m, edge_index, W1, b1, g1, be1, W2, b2, g2, be2):
    src = edge_index[0].astype(jnp.int32)
    dst = edge_index[1].astype(jnp.int32)
    Wa, Wb = W1[:, :D], W1[:, D:]

    h1 = jax.ops.segment_sum(m, dst, num_segments=N_NODES)
    hs = jax.ops.segment_sum(m, src, num_segments=N_NODES)
    c = jax.ops.segment_sum(jnp.ones((N_EDGES,), jnp.float32), src,
                            num_segments=N_NODES)
    G, sm = _tg(m)
    sm = sm[0]

    hp = jax.lax.Precision.HIGHEST
    p = jnp.dot(h1, Wa.T, precision=hp)
    Qsum = jnp.dot(hs, Wb.T, precision=hp)
    colsum_q = jnp.dot(sm, Wb.T, precision=hp)
    colsumsq_q = jnp.sum(jnp.dot(Wb, G, precision=hp) * Wb, axis=1)
    S_r = jnp.dot(c, p, precision=hp)
    S_rr = jnp.dot(c, p * p, precision=hp)
    X = jnp.sum(p * Qsum, axis=0)
    E = float(N_EDGES)
    sum_z = colsum_q + S_r + E * b1
    sumsq_z = (colsumsq_q + S_rr + E * b1 * b1 + 2 * X
               + 2 * b1 * colsum_q + 2 * b1 * S_r)
    mean = sum_z / E
    var = sumsq_z / E - mean * mean
    a = g1 / jnp.sqrt(var + 1e-5)
    e0 = (a * (b1 - mean) + be1)[None, :]
    pa = p * a
    wbp_t = (Wb * a[:, None]).T

    r = pa[src]
    y = _t2(m, wbp_t, r, e0)
    h2 = jax.ops.segment_sum(y, dst, num_segments=N_NODES)
    return h2


# speedup vs baseline: 1.6970x; 1.6970x over previous
import functools
import jax
import jax.numpy as jnp
from jax import lax
from jax.experimental import pallas as pl
from jax.experimental.pallas import tpu as pltpu
from jax.experimental.pallas import tpu_sc as plsc

N_NODES = 10000
N_EDGES = 320000
D = 128
BLK = 2560
assert N_EDGES % BLK == 0

NC = 2
NS = 16
CH = 80
NBUF = 2

STRIPE = 624
TAIL0 = NS * STRIPE
TAILN = N_NODES - TAIL0


def _zero_rows(buf, nrows):
    zv = jnp.zeros((16,), jnp.float32)

    def body(i, _):
        for c in range(D // 16):
            buf[i, pl.ds(c * 16, 16)] = zv
        return 0

    lax.fori_loop(0, nrows, body, 0)


def _zero_spmem_stripe(sh, row0, nrows, zbuf, zrows):
    full, rem = nrows // zrows, nrows % zrows
    for k in range(full):
        pltpu.sync_copy(zbuf.at[pl.ds(0, zrows)],
                        sh.at[pl.ds(row0 + k * zrows, zrows)])
    if rem:
        pltpu.sync_copy(zbuf.at[pl.ds(0, rem)],
                        sh.at[pl.ds(row0 + full * zrows, rem)])


def _s1_body(m_hbm, dst_hbm, src_hbm, out_hbm,
             buf0, buf1, idx0, idx1, sh_h,
             sem_i0, sem_i1, sem_d0, sem_d1):
    cid = lax.axis_index("c")
    sid = lax.axis_index("s")
    epw = N_EDGES // NS
    nchunk = epw // CH
    ebase = sid * epw

    bufs = (buf0, buf1)
    idxs = (idx0, idx1)
    sem_is = (sem_i0, sem_i1)
    sem_ds = (sem_d0, sem_d1)

    _zero_rows(buf0, CH)
    r0 = sid * STRIPE
    _zero_spmem_stripe(sh_h, r0, STRIPE, buf0, CH)

    @pl.when(sid == NS - 1)
    def _():
        pltpu.sync_copy(buf0.at[pl.ds(0, TAILN)], sh_h.at[pl.ds(TAIL0, TAILN)])

    plsc.subcore_barrier()

    def run_pipeline(idx_hbm):
        def issue(j, b):
            off = ebase + j * CH
            pltpu.async_copy(idx_hbm.at[pl.ds(off, CH)], idxs[b], sem_is[b])
            pltpu.async_copy(m_hbm.at[pl.ds(off, CH)], bufs[b], sem_ds[b])

        for b in range(NBUF):
            issue(b, b)

        def round_body(g, _):
            for b in range(NBUF):
                j = g + b
                pltpu.make_async_copy(idx_hbm.at[pl.ds(0, CH)],
                                      idxs[b], sem_is[b]).wait()
                pltpu.make_async_copy(m_hbm.at[pl.ds(0, CH)],
                                      bufs[b], sem_ds[b]).wait()
                pltpu.sync_copy(bufs[b], sh_h.at[idxs[b]], add=True)

                @pl.when(j + NBUF < nchunk)
                def _():
                    issue(j + NBUF, b)
            return 0

        lax.fori_loop(0, nchunk // NBUF,
                      lambda g, x: round_body(g * NBUF, x), 0, unroll=False)
        for b in range(nchunk % NBUF):
            pltpu.make_async_copy(idx_hbm.at[pl.ds(0, CH)],
                                  idxs[b], sem_is[b]).wait()
            pltpu.make_async_copy(m_hbm.at[pl.ds(0, CH)],
                                  bufs[b], sem_ds[b]).wait()
            pltpu.sync_copy(bufs[b], sh_h.at[idxs[b]], add=True)

    @pl.when(cid == 0)
    def _():
        run_pipeline(dst_hbm)

    @pl.when(cid == 1)
    def _():
        run_pipeline(src_hbm)

    plsc.subcore_barrier()

    pltpu.sync_copy(sh_h.at[pl.ds(r0, STRIPE)],
                    out_hbm.at[cid, pl.ds(r0, STRIPE)])

    @pl.when(sid == NS - 1)
    def _():
        pltpu.sync_copy(sh_h.at[pl.ds(TAIL0, TAILN)],
                        out_hbm.at[cid, pl.ds(TAIL0, TAILN)])


def _s1(m, ei):
    mesh = plsc.VectorSubcoreMesh(core_axis_name="c", subcore_axis_name="s")
    f = functools.partial(
        pl.kernel,
        mesh=mesh,
        out_type=jax.ShapeDtypeStruct((NC, N_NODES, D), jnp.float32),
        scratch_types=[
            pltpu.VMEM((CH, D), jnp.float32),
            pltpu.VMEM((CH, D), jnp.float32),
            pltpu.VMEM((CH,), jnp.int32),
            pltpu.VMEM((CH,), jnp.int32),
            pltpu.VMEM_SHARED((N_NODES, D), jnp.float32),
            pltpu.SemaphoreType.DMA,
            pltpu.SemaphoreType.DMA,
            pltpu.SemaphoreType.DMA,
            pltpu.SemaphoreType.DMA,
        ],
    )(_s1_body)
    return f(m, ei[1], ei[0])


def _scnt_body(src_hbm, cnt_hbm, idx0, idx1, ones_v, zc_v, sh_c,
               sem_i0, sem_i1):
    cid = lax.axis_index("c")
    sid = lax.axis_index("s")
    epw = N_EDGES // (NC * NS)
    nchunk = epw // CH
    ebase = (cid * NS + sid) * epw

    idxs = (idx0, idx1)
    sem_is = (sem_i0, sem_i1)

    ov = jnp.ones((16,), jnp.float32)

    def fill(i, _):
        for cc in range(D // 16):
            ones_v[i, pl.ds(cc * 16, 16)] = ov
        return 0
    lax.fori_loop(0, CH, fill, 0)
    _zero_rows(zc_v, CH)

    r0 = sid * STRIPE
    _zero_spmem_stripe(sh_c, r0, STRIPE, zc_v, CH)

    @pl.when(sid == NS - 1)
    def _():
        pltpu.sync_copy(zc_v.at[pl.ds(0, TAILN)], sh_c.at[pl.ds(TAIL0, TAILN)])

    plsc.subcore_barrier()

    def issue(j, b):
        pltpu.async_copy(src_hbm.at[pl.ds(ebase + j * CH, CH)],
                         idxs[b], sem_is[b])

    for b in range(NBUF):
        issue(b, b)

    def round_body(g, _):
        for b in range(NBUF):
            j = g + b
            pltpu.make_async_copy(src_hbm.at[pl.ds(0, CH)],
                                  idxs[b], sem_is[b]).wait()
            pltpu.sync_copy(ones_v, sh_c.at[idxs[b]], add=True)

            @pl.when(j + NBUF < nchunk)
            def _():
                issue(j + NBUF, b)
        return 0

    lax.fori_loop(0, nchunk // NBUF,
                  lambda g, x: round_body(g * NBUF, x), 0, unroll=False)
    for b in range(nchunk % NBUF):
        pltpu.make_async_copy(src_hbm.at[pl.ds(0, CH)],
                              idxs[b], sem_is[b]).wait()
        pltpu.sync_copy(ones_v, sh_c.at[idxs[b]], add=True)

    plsc.subcore_barrier()
    pltpu.sync_copy(sh_c.at[pl.ds(r0, STRIPE)],
                    cnt_hbm.at[cid, pl.ds(r0, STRIPE)])

    @pl.when(sid == NS - 1)
    def _():
        pltpu.sync_copy(sh_c.at[pl.ds(TAIL0, TAILN)],
                        cnt_hbm.at[cid, pl.ds(TAIL0, TAILN)])


def _scnt(src):
    mesh = plsc.VectorSubcoreMesh(core_axis_name="c", subcore_axis_name="s")
    f = functools.partial(
        pl.kernel,
        mesh=mesh,
        out_type=jax.ShapeDtypeStruct((NC, N_NODES, D), jnp.float32),
        scratch_types=[
            pltpu.VMEM((CH,), jnp.int32),
            pltpu.VMEM((CH,), jnp.int32),
            pltpu.VMEM((CH, D), jnp.float32),
            pltpu.VMEM((CH, D), jnp.float32),
            pltpu.VMEM_SHARED((N_NODES, D), jnp.float32),
            pltpu.SemaphoreType.DMA,
            pltpu.SemaphoreType.DMA,
        ],
    )(_scnt_body)
    return f(src)


def _t2_body(m_ref, w_ref, r_ref, e0_ref, y_ref):
    acc = jnp.dot(m_ref[...], w_ref[...], preferred_element_type=jnp.float32,
                  precision=jax.lax.Precision.HIGHEST)
    y_ref[...] = jnp.maximum(acc + r_ref[...] + e0_ref[...], 0.0)


def _t2(m, wbp_t, r, e0):
    grid = (N_EDGES // BLK,)
    return pl.pallas_call(
        _t2_body,
        grid=grid,
        in_specs=[
            pl.BlockSpec((BLK, D), lambda i: (i, 0)),
            pl.BlockSpec((D, D), lambda i: (0, 0)),
            pl.BlockSpec((BLK, D), lambda i: (i, 0)),
            pl.BlockSpec((1, D), lambda i: (0, 0)),
        ],
        out_specs=pl.BlockSpec((BLK, D), lambda i: (i, 0)),
        out_shape=jax.ShapeDtypeStruct((N_EDGES, D), jnp.float32),
    )(m, wbp_t, r, e0)


def _tg_body(m_ref, g_ref, s_ref):
    @pl.when(pl.program_id(0) == 0)
    def _():
        g_ref[...] = jnp.zeros_like(g_ref)
        s_ref[...] = jnp.zeros_like(s_ref)

    blk = m_ref[...]
    g_ref[...] += jnp.dot(blk.T, blk, preferred_element_type=jnp.float32,
                          precision=jax.lax.Precision.HIGHEST)
    s_ref[...] += jnp.sum(blk, axis=0, keepdims=True)


def _tg(m):
    grid = (N_EDGES // BLK,)
    return pl.pallas_call(
        _tg_body,
        grid=grid,
        in_specs=[pl.BlockSpec((BLK, D), lambda i: (i, 0))],
        out_specs=[
            pl.BlockSpec((D, D), lambda i: (0, 0)),
            pl.BlockSpec((1, D), lambda i: (0, 0)),
        ],
        out_shape=[
            jax.ShapeDtypeStruct((D, D), jnp.float32),
            jax.ShapeDtypeStruct((1, D), jnp.float32),
        ],
    )(m)


def kernel(m, edge_index, W1, b1, g1, be1, W2, b2, g2, be2):
    ei = edge_index.astype(jnp.int32)
    src = ei[0]
    dst = ei[1]
    Wa, Wb = W1[:, :D], W1[:, D:]

    hh = _s1(m, ei)
    h1, hs = hh[0], hh[1]
    cnt16 = _scnt(src)
    c = cnt16[0, :, 0] + cnt16[1, :, 0]
    G, sm = _tg(m)
    sm = sm[0]

    hp = jax.lax.Precision.HIGHEST
    p = jnp.dot(h1, Wa.T, precision=hp)
    Qsum = jnp.dot(hs, Wb.T, precision=hp)
    colsum_q = jnp.dot(sm, Wb.T, precision=hp)
    colsumsq_q = jnp.sum(jnp.dot(Wb, G, precision=hp) * Wb, axis=1)
    S_r = jnp.dot(c, p, precision=hp)
    S_rr = jnp.dot(c, p * p, precision=hp)
    X = jnp.sum(p * Qsum, axis=0)
    E = float(N_EDGES)
    sum_z = colsum_q + S_r + E * b1
    sumsq_z = (colsumsq_q + S_rr + E * b1 * b1 + 2 * X
               + 2 * b1 * colsum_q + 2 * b1 * S_r)
    mean = sum_z / E
    var = sumsq_z / E - mean * mean
    a = g1 / jnp.sqrt(var + 1e-5)
    e0 = (a * (b1 - mean) + be1)[None, :]
    pa = p * a
    wbp_t = (Wb * a[:, None]).T

    r = pa[src]
    y = _t2(m, wbp_t, r, e0)
    h2 = jax.ops.segment_sum(y, dst, num_segments=N_NODES)
    return h2


# speedup vs baseline: 4.8355x; 2.8494x over previous
import functools
import jax
import jax.numpy as jnp
from jax import lax
from jax.experimental import pallas as pl
from jax.experimental.pallas import tpu as pltpu
from jax.experimental.pallas import tpu_sc as plsc

N_NODES = 10000
N_EDGES = 320000
D = 128
BLK = 2560
assert N_EDGES % BLK == 0

NC = 2
NS = 16
CH = 80
NBUF = 2

STRIPE = 624
TAIL0 = NS * STRIPE
TAILN = N_NODES - TAIL0


def _zero_rows(buf, nrows):
    zv = jnp.zeros((16,), jnp.float32)

    def body(i, _):
        for c in range(D // 16):
            buf[i, pl.ds(c * 16, 16)] = zv
        return 0

    lax.fori_loop(0, nrows, body, 0)


def _zero_spmem_stripe(sh, row0, nrows, zbuf, zrows):
    full, rem = nrows // zrows, nrows % zrows
    for k in range(full):
        pltpu.sync_copy(zbuf.at[pl.ds(0, zrows)],
                        sh.at[pl.ds(row0 + k * zrows, zrows)])
    if rem:
        pltpu.sync_copy(zbuf.at[pl.ds(0, rem)],
                        sh.at[pl.ds(row0 + full * zrows, rem)])


def _s1_body(m_hbm, dst_hbm, src_hbm, out_hbm,
             buf0, buf1, idx0, idx1, sh_h,
             sem_i0, sem_i1, sem_d0, sem_d1):
    cid = lax.axis_index("c")
    sid = lax.axis_index("s")
    epw = N_EDGES // NS
    nchunk = epw // CH
    ebase = sid * epw

    bufs = (buf0, buf1)
    idxs = (idx0, idx1)
    sem_is = (sem_i0, sem_i1)
    sem_ds = (sem_d0, sem_d1)

    _zero_rows(buf0, CH)
    r0 = sid * STRIPE
    _zero_spmem_stripe(sh_h, r0, STRIPE, buf0, CH)

    @pl.when(sid == NS - 1)
    def _():
        pltpu.sync_copy(buf0.at[pl.ds(0, TAILN)], sh_h.at[pl.ds(TAIL0, TAILN)])

    plsc.subcore_barrier()

    def run_pipeline(idx_hbm):
        def issue(j, b):
            off = ebase + j * CH
            pltpu.async_copy(idx_hbm.at[pl.ds(off, CH)], idxs[b], sem_is[b])
            pltpu.async_copy(m_hbm.at[pl.ds(off, CH)], bufs[b], sem_ds[b])

        for b in range(NBUF):
            issue(b, b)

        def round_body(g, _):
            for b in range(NBUF):
                j = g + b
                pltpu.make_async_copy(idx_hbm.at[pl.ds(0, CH)],
                                      idxs[b], sem_is[b]).wait()
                pltpu.make_async_copy(m_hbm.at[pl.ds(0, CH)],
                                      bufs[b], sem_ds[b]).wait()
                pltpu.sync_copy(bufs[b], sh_h.at[idxs[b]], add=True)

                @pl.when(j + NBUF < nchunk)
                def _():
                    issue(j + NBUF, b)
            return 0

        lax.fori_loop(0, nchunk // NBUF,
                      lambda g, x: round_body(g * NBUF, x), 0, unroll=False)
        for b in range(nchunk % NBUF):
            pltpu.make_async_copy(idx_hbm.at[pl.ds(0, CH)],
                                  idxs[b], sem_is[b]).wait()
            pltpu.make_async_copy(m_hbm.at[pl.ds(0, CH)],
                                  bufs[b], sem_ds[b]).wait()
            pltpu.sync_copy(bufs[b], sh_h.at[idxs[b]], add=True)

    @pl.when(cid == 0)
    def _():
        run_pipeline(dst_hbm)

    @pl.when(cid == 1)
    def _():
        run_pipeline(src_hbm)

    plsc.subcore_barrier()

    pltpu.sync_copy(sh_h.at[pl.ds(r0, STRIPE)],
                    out_hbm.at[cid, pl.ds(r0, STRIPE)])

    @pl.when(sid == NS - 1)
    def _():
        pltpu.sync_copy(sh_h.at[pl.ds(TAIL0, TAILN)],
                        out_hbm.at[cid, pl.ds(TAIL0, TAILN)])


def _s1(m, ei):
    mesh = plsc.VectorSubcoreMesh(core_axis_name="c", subcore_axis_name="s")
    f = functools.partial(
        pl.kernel,
        mesh=mesh,
        out_type=jax.ShapeDtypeStruct((NC, N_NODES, D), jnp.float32),
        scratch_types=[
            pltpu.VMEM((CH, D), jnp.float32),
            pltpu.VMEM((CH, D), jnp.float32),
            pltpu.VMEM((CH,), jnp.int32),
            pltpu.VMEM((CH,), jnp.int32),
            pltpu.VMEM_SHARED((N_NODES, D), jnp.float32),
            pltpu.SemaphoreType.DMA,
            pltpu.SemaphoreType.DMA,
            pltpu.SemaphoreType.DMA,
            pltpu.SemaphoreType.DMA,
        ],
    )(_s1_body)
    return f(m, ei[1], ei[0])


def _scnt_body(src_hbm, cnt_hbm, idx0, idx1, ones_v, zc_v, sh_c,
               sem_i0, sem_i1):
    cid = lax.axis_index("c")
    sid = lax.axis_index("s")
    epw = N_EDGES // (NC * NS)
    nchunk = epw // CH
    ebase = (cid * NS + sid) * epw

    idxs = (idx0, idx1)
    sem_is = (sem_i0, sem_i1)

    ov = jnp.ones((16,), jnp.float32)

    def fill(i, _):
        for cc in range(D // 16):
            ones_v[i, pl.ds(cc * 16, 16)] = ov
        return 0
    lax.fori_loop(0, CH, fill, 0)
    _zero_rows(zc_v, CH)

    r0 = sid * STRIPE
    _zero_spmem_stripe(sh_c, r0, STRIPE, zc_v, CH)

    @pl.when(sid == NS - 1)
    def _():
        pltpu.sync_copy(zc_v.at[pl.ds(0, TAILN)], sh_c.at[pl.ds(TAIL0, TAILN)])

    plsc.subcore_barrier()

    def issue(j, b):
        pltpu.async_copy(src_hbm.at[pl.ds(ebase + j * CH, CH)],
                         idxs[b], sem_is[b])

    for b in range(NBUF):
        issue(b, b)

    def round_body(g, _):
        for b in range(NBUF):
            j = g + b
            pltpu.make_async_copy(src_hbm.at[pl.ds(0, CH)],
                                  idxs[b], sem_is[b]).wait()
            pltpu.sync_copy(ones_v, sh_c.at[idxs[b]], add=True)

            @pl.when(j + NBUF < nchunk)
            def _():
                issue(j + NBUF, b)
        return 0

    lax.fori_loop(0, nchunk // NBUF,
                  lambda g, x: round_body(g * NBUF, x), 0, unroll=False)
    for b in range(nchunk % NBUF):
        pltpu.make_async_copy(src_hbm.at[pl.ds(0, CH)],
                              idxs[b], sem_is[b]).wait()
        pltpu.sync_copy(ones_v, sh_c.at[idxs[b]], add=True)

    plsc.subcore_barrier()
    pltpu.sync_copy(sh_c.at[pl.ds(r0, STRIPE)],
                    cnt_hbm.at[cid, pl.ds(r0, STRIPE)])

    @pl.when(sid == NS - 1)
    def _():
        pltpu.sync_copy(sh_c.at[pl.ds(TAIL0, TAILN)],
                        cnt_hbm.at[cid, pl.ds(TAIL0, TAILN)])


def _scnt(src):
    mesh = plsc.VectorSubcoreMesh(core_axis_name="c", subcore_axis_name="s")
    f = functools.partial(
        pl.kernel,
        mesh=mesh,
        out_type=jax.ShapeDtypeStruct((NC, N_NODES, D), jnp.float32),
        scratch_types=[
            pltpu.VMEM((CH,), jnp.int32),
            pltpu.VMEM((CH,), jnp.int32),
            pltpu.VMEM((CH, D), jnp.float32),
            pltpu.VMEM((CH, D), jnp.float32),
            pltpu.VMEM_SHARED((N_NODES, D), jnp.float32),
            pltpu.SemaphoreType.DMA,
            pltpu.SemaphoreType.DMA,
        ],
    )(_scnt_body)
    return f(src)


def _sgather_body(pa_hbm, src_hbm, r_hbm, idx0, idx1, buf0, buf1, sh_p,
                  sem_i0, sem_i1, sem_w0, sem_w1):
    cid = lax.axis_index("c")
    sid = lax.axis_index("s")
    epw = N_EDGES // (NC * NS)
    nchunk = epw // CH
    ebase = (cid * NS + sid) * epw

    idxs = (idx0, idx1)
    bufs = (buf0, buf1)
    sem_is = (sem_i0, sem_i1)
    sem_ws = (sem_w0, sem_w1)

    r0 = sid * STRIPE
    pltpu.sync_copy(pa_hbm.at[pl.ds(r0, STRIPE)], sh_p.at[pl.ds(r0, STRIPE)])

    @pl.when(sid == NS - 1)
    def _():
        pltpu.sync_copy(pa_hbm.at[pl.ds(TAIL0, TAILN)],
                        sh_p.at[pl.ds(TAIL0, TAILN)])

    plsc.subcore_barrier()

    def issue(j, b):
        pltpu.async_copy(src_hbm.at[pl.ds(ebase + j * CH, CH)],
                         idxs[b], sem_is[b])

    for b in range(NBUF):
        issue(b, b)

    def chunk_step(j, b):
        @pl.when(j >= NBUF)
        def _():
            pltpu.make_async_copy(bufs[b], r_hbm.at[pl.ds(0, CH)],
                                  sem_ws[b]).wait()
        pltpu.make_async_copy(src_hbm.at[pl.ds(0, CH)],
                              idxs[b], sem_is[b]).wait()
        pltpu.sync_copy(sh_p.at[idxs[b]], bufs[b])
        pltpu.async_copy(bufs[b], r_hbm.at[pl.ds(ebase + j * CH, CH)],
                         sem_ws[b])

        @pl.when(j + NBUF < nchunk)
        def _():
            issue(j + NBUF, b)

    def round_body(g, _):
        for b in range(NBUF):
            chunk_step(g + b, b)
        return 0

    lax.fori_loop(0, nchunk // NBUF,
                  lambda g, x: round_body(g * NBUF, x), 0, unroll=False)
    for b in range(nchunk % NBUF):
        chunk_step((nchunk // NBUF) * NBUF + b, b)
    for b in range(NBUF):
        pltpu.make_async_copy(bufs[b], r_hbm.at[pl.ds(0, CH)],
                              sem_ws[b]).wait()


def _sgather(pa, src):
    mesh = plsc.VectorSubcoreMesh(core_axis_name="c", subcore_axis_name="s")
    f = functools.partial(
        pl.kernel,
        mesh=mesh,
        out_type=jax.ShapeDtypeStruct((N_EDGES, D), jnp.float32),
        scratch_types=[
            pltpu.VMEM((CH,), jnp.int32),
            pltpu.VMEM((CH,), jnp.int32),
            pltpu.VMEM((CH, D), jnp.float32),
            pltpu.VMEM((CH, D), jnp.float32),
            pltpu.VMEM_SHARED((N_NODES, D), jnp.float32),
            pltpu.SemaphoreType.DMA,
            pltpu.SemaphoreType.DMA,
            pltpu.SemaphoreType.DMA,
            pltpu.SemaphoreType.DMA,
        ],
    )(_sgather_body)
    return f(pa, src)


def _s2_body(y_hbm, dst_hbm, out_hbm, buf0, buf1, idx0, idx1, sh_h,
             sem_i0, sem_i1, sem_d0, sem_d1):
    cid = lax.axis_index("c")
    sid = lax.axis_index("s")
    epw = N_EDGES // (NC * NS)
    nchunk = epw // CH
    ebase = (cid * NS + sid) * epw

    bufs = (buf0, buf1)
    idxs = (idx0, idx1)
    sem_is = (sem_i0, sem_i1)
    sem_ds = (sem_d0, sem_d1)

    _zero_rows(buf0, CH)
    r0 = sid * STRIPE
    _zero_spmem_stripe(sh_h, r0, STRIPE, buf0, CH)

    @pl.when(sid == NS - 1)
    def _():
        pltpu.sync_copy(buf0.at[pl.ds(0, TAILN)], sh_h.at[pl.ds(TAIL0, TAILN)])

    plsc.subcore_barrier()

    def issue(j, b):
        off = ebase + j * CH
        pltpu.async_copy(dst_hbm.at[pl.ds(off, CH)], idxs[b], sem_is[b])
        pltpu.async_copy(y_hbm.at[pl.ds(off, CH)], bufs[b], sem_ds[b])

    for b in range(NBUF):
        issue(b, b)

    def chunk_step(j, b):
        pltpu.make_async_copy(dst_hbm.at[pl.ds(0, CH)],
                              idxs[b], sem_is[b]).wait()
        pltpu.make_async_copy(y_hbm.at[pl.ds(0, CH)],
                              bufs[b], sem_ds[b]).wait()
        pltpu.sync_copy(bufs[b], sh_h.at[idxs[b]], add=True)

        @pl.when(j + NBUF < nchunk)
        def _():
            issue(j + NBUF, b)

    def round_body(g, _):
        for b in range(NBUF):
            chunk_step(g + b, b)
        return 0

    lax.fori_loop(0, nchunk // NBUF,
                  lambda g, x: round_body(g * NBUF, x), 0, unroll=False)
    for b in range(nchunk % NBUF):
        chunk_step((nchunk // NBUF) * NBUF + b, b)

    plsc.subcore_barrier()
    pltpu.sync_copy(sh_h.at[pl.ds(r0, STRIPE)],
                    out_hbm.at[cid, pl.ds(r0, STRIPE)])

    @pl.when(sid == NS - 1)
    def _():
        pltpu.sync_copy(sh_h.at[pl.ds(TAIL0, TAILN)],
                        out_hbm.at[cid, pl.ds(TAIL0, TAILN)])


def _s2(y, dst):
    mesh = plsc.VectorSubcoreMesh(core_axis_name="c", subcore_axis_name="s")
    f = functools.partial(
        pl.kernel,
        mesh=mesh,
        out_type=jax.ShapeDtypeStruct((NC, N_NODES, D), jnp.float32),
        scratch_types=[
            pltpu.VMEM((CH, D), jnp.float32),
            pltpu.VMEM((CH, D), jnp.float32),
            pltpu.VMEM((CH,), jnp.int32),
            pltpu.VMEM((CH,), jnp.int32),
            pltpu.VMEM_SHARED((N_NODES, D), jnp.float32),
            pltpu.SemaphoreType.DMA,
            pltpu.SemaphoreType.DMA,
            pltpu.SemaphoreType.DMA,
            pltpu.SemaphoreType.DMA,
        ],
    )(_s2_body)
    return f(y, dst)


NBLK = 2000


def _tadd_body(a_ref, b_ref, o_ref):
    o_ref[...] = a_ref[0] + b_ref[0]


def _tadd(parts):
    grid = (N_NODES // NBLK,)
    return pl.pallas_call(
        _tadd_body,
        grid=grid,
        in_specs=[
            pl.BlockSpec((1, NBLK, D), lambda i: (0, i, 0)),
            pl.BlockSpec((1, NBLK, D), lambda i: (1, i, 0)),
        ],
        out_specs=pl.BlockSpec((NBLK, D), lambda i: (i, 0)),
        out_shape=jax.ShapeDtypeStruct((N_NODES, D), jnp.float32),
    )(parts, parts)


def _t2_body(m_ref, w_ref, r_ref, e0_ref, y_ref):
    acc = jnp.dot(m_ref[...], w_ref[...], preferred_element_type=jnp.float32,
                  precision=jax.lax.Precision.HIGHEST)
    y_ref[...] = jnp.maximum(acc + r_ref[...] + e0_ref[...], 0.0)


def _t2(m, wbp_t, r, e0):
    grid = (N_EDGES // BLK,)
    return pl.pallas_call(
        _t2_body,
        grid=grid,
        in_specs=[
            pl.BlockSpec((BLK, D), lambda i: (i, 0)),
            pl.BlockSpec((D, D), lambda i: (0, 0)),
            pl.BlockSpec((BLK, D), lambda i: (i, 0)),
            pl.BlockSpec((1, D), lambda i: (0, 0)),
        ],
        out_specs=pl.BlockSpec((BLK, D), lambda i: (i, 0)),
        out_shape=jax.ShapeDtypeStruct((N_EDGES, D), jnp.float32),
    )(m, wbp_t, r, e0)


def _tg_body(m_ref, g_ref, s_ref):
    @pl.when(pl.program_id(0) == 0)
    def _():
        g_ref[...] = jnp.zeros_like(g_ref)
        s_ref[...] = jnp.zeros_like(s_ref)

    blk = m_ref[...]
    g_ref[...] += jnp.dot(blk.T, blk, preferred_element_type=jnp.float32,
                          precision=jax.lax.Precision.HIGHEST)
    s_ref[...] += jnp.sum(blk, axis=0, keepdims=True)


def _tg(m):
    grid = (N_EDGES // BLK,)
    return pl.pallas_call(
        _tg_body,
        grid=grid,
        in_specs=[pl.BlockSpec((BLK, D), lambda i: (i, 0))],
        out_specs=[
            pl.BlockSpec((D, D), lambda i: (0, 0)),
            pl.BlockSpec((1, D), lambda i: (0, 0)),
        ],
        out_shape=[
            jax.ShapeDtypeStruct((D, D), jnp.float32),
            jax.ShapeDtypeStruct((1, D), jnp.float32),
        ],
    )(m)


def kernel(m, edge_index, W1, b1, g1, be1, W2, b2, g2, be2):
    ei = edge_index.astype(jnp.int32)
    src = ei[0]
    dst = ei[1]
    Wa, Wb = W1[:, :D], W1[:, D:]

    hh = _s1(m, ei)
    h1, hs = hh[0], hh[1]
    cnt16 = _scnt(src)
    c = cnt16[0, :, 0] + cnt16[1, :, 0]
    G, sm = _tg(m)
    sm = sm[0]

    hp = jax.lax.Precision.HIGHEST
    p = jnp.dot(h1, Wa.T, precision=hp)
    Qsum = jnp.dot(hs, Wb.T, precision=hp)
    colsum_q = jnp.dot(sm, Wb.T, precision=hp)
    colsumsq_q = jnp.sum(jnp.dot(Wb, G, precision=hp) * Wb, axis=1)
    S_r = jnp.dot(c, p, precision=hp)
    S_rr = jnp.dot(c, p * p, precision=hp)
    X = jnp.sum(p * Qsum, axis=0)
    E = float(N_EDGES)
    sum_z = colsum_q + S_r + E * b1
    sumsq_z = (colsumsq_q + S_rr + E * b1 * b1 + 2 * X
               + 2 * b1 * colsum_q + 2 * b1 * S_r)
    mean = sum_z / E
    var = sumsq_z / E - mean * mean
    a = g1 / jnp.sqrt(var + 1e-5)
    e0 = (a * (b1 - mean) + be1)[None, :]
    pa = p * a
    wbp_t = (Wb * a[:, None]).T

    r = _sgather(pa, src)
    y = _t2(m, wbp_t, r, e0)
    parts = _s2(y, dst)
    h2 = _tadd(parts)
    return h2


# speedup vs baseline: 4.8370x; 1.0003x over previous
import functools
import jax
import jax.numpy as jnp
from jax import lax
from jax.experimental import pallas as pl
from jax.experimental.pallas import tpu as pltpu
from jax.experimental.pallas import tpu_sc as plsc

N_NODES = 10000
N_EDGES = 320000
D = 128
BLK = 2560
assert N_EDGES % BLK == 0

NC = 2
NS = 16
CH = 80
NBUF = 2

STRIPE = 624
TAIL0 = NS * STRIPE
TAILN = N_NODES - TAIL0


def _zero_rows(buf, nrows):
    zv = jnp.zeros((16,), jnp.float32)

    def body(i, _):
        for c in range(D // 16):
            buf[i, pl.ds(c * 16, 16)] = zv
        return 0

    lax.fori_loop(0, nrows, body, 0)


def _zero_spmem_stripe(sh, row0, nrows, zbuf, zrows):
    full, rem = nrows // zrows, nrows % zrows
    for k in range(full):
        pltpu.sync_copy(zbuf.at[pl.ds(0, zrows)],
                        sh.at[pl.ds(row0 + k * zrows, zrows)])
    if rem:
        pltpu.sync_copy(zbuf.at[pl.ds(0, rem)],
                        sh.at[pl.ds(row0 + full * zrows, rem)])


def _s1_body(m_hbm, dst_hbm, src_hbm, out_hbm,
             buf0, buf1, idx0, idx1, sh_h,
             sem_i0, sem_i1, sem_d0, sem_d1):
    cid = lax.axis_index("c")
    sid = lax.axis_index("s")
    epw = N_EDGES // NS
    nchunk = epw // CH
    ebase = sid * epw

    bufs = (buf0, buf1)
    idxs = (idx0, idx1)
    sem_is = (sem_i0, sem_i1)
    sem_ds = (sem_d0, sem_d1)

    _zero_rows(buf0, CH)
    r0 = sid * STRIPE
    _zero_spmem_stripe(sh_h, r0, STRIPE, buf0, CH)

    @pl.when(sid == NS - 1)
    def _():
        pltpu.sync_copy(buf0.at[pl.ds(0, TAILN)], sh_h.at[pl.ds(TAIL0, TAILN)])

    plsc.subcore_barrier()

    def run_pipeline(idx_hbm):
        def issue(j, b):
            off = ebase + j * CH
            pltpu.async_copy(idx_hbm.at[pl.ds(off, CH)], idxs[b], sem_is[b])
            pltpu.async_copy(m_hbm.at[pl.ds(off, CH)], bufs[b], sem_ds[b])

        for b in range(NBUF):
            issue(b, b)

        def round_body(g, _):
            for b in range(NBUF):
                j = g + b
                pltpu.make_async_copy(idx_hbm.at[pl.ds(0, CH)],
                                      idxs[b], sem_is[b]).wait()
                pltpu.make_async_copy(m_hbm.at[pl.ds(0, CH)],
                                      bufs[b], sem_ds[b]).wait()
                pltpu.sync_copy(bufs[b], sh_h.at[idxs[b]], add=True)

                @pl.when(j + NBUF < nchunk)
                def _():
                    issue(j + NBUF, b)
            return 0

        lax.fori_loop(0, nchunk // NBUF,
                      lambda g, x: round_body(g * NBUF, x), 0, unroll=False)
        for b in range(nchunk % NBUF):
            pltpu.make_async_copy(idx_hbm.at[pl.ds(0, CH)],
                                  idxs[b], sem_is[b]).wait()
            pltpu.make_async_copy(m_hbm.at[pl.ds(0, CH)],
                                  bufs[b], sem_ds[b]).wait()
            pltpu.sync_copy(bufs[b], sh_h.at[idxs[b]], add=True)

    @pl.when(cid == 0)
    def _():
        run_pipeline(dst_hbm)

    @pl.when(cid == 1)
    def _():
        run_pipeline(src_hbm)

    plsc.subcore_barrier()

    pltpu.sync_copy(sh_h.at[pl.ds(r0, STRIPE)],
                    out_hbm.at[cid, pl.ds(r0, STRIPE)])

    @pl.when(sid == NS - 1)
    def _():
        pltpu.sync_copy(sh_h.at[pl.ds(TAIL0, TAILN)],
                        out_hbm.at[cid, pl.ds(TAIL0, TAILN)])


def _s1(m, ei):
    mesh = plsc.VectorSubcoreMesh(core_axis_name="c", subcore_axis_name="s")
    f = functools.partial(
        pl.kernel,
        mesh=mesh,
        out_type=jax.ShapeDtypeStruct((NC, N_NODES, D), jnp.float32),
        scratch_types=[
            pltpu.VMEM((CH, D), jnp.float32),
            pltpu.VMEM((CH, D), jnp.float32),
            pltpu.VMEM((CH,), jnp.int32),
            pltpu.VMEM((CH,), jnp.int32),
            pltpu.VMEM_SHARED((N_NODES, D), jnp.float32),
            pltpu.SemaphoreType.DMA,
            pltpu.SemaphoreType.DMA,
            pltpu.SemaphoreType.DMA,
            pltpu.SemaphoreType.DMA,
        ],
    )(_s1_body)
    return f(m, ei[1], ei[0])


def _scnt_body(src_hbm, cnt_hbm, idx0, idx1, ones_v, zc_v, sh_c,
               sem_i0, sem_i1):
    cid = lax.axis_index("c")
    sid = lax.axis_index("s")
    epw = N_EDGES // (NC * NS)
    nchunk = epw // CH
    ebase = (cid * NS + sid) * epw

    idxs = (idx0, idx1)
    sem_is = (sem_i0, sem_i1)

    ov = jnp.ones((16,), jnp.float32)

    def fill(i, _):
        for cc in range(D // 16):
            ones_v[i, pl.ds(cc * 16, 16)] = ov
        return 0
    lax.fori_loop(0, CH, fill, 0)
    _zero_rows(zc_v, CH)

    r0 = sid * STRIPE
    _zero_spmem_stripe(sh_c, r0, STRIPE, zc_v, CH)

    @pl.when(sid == NS - 1)
    def _():
        pltpu.sync_copy(zc_v.at[pl.ds(0, TAILN)], sh_c.at[pl.ds(TAIL0, TAILN)])

    plsc.subcore_barrier()

    def issue(j, b):
        pltpu.async_copy(src_hbm.at[pl.ds(ebase + j * CH, CH)],
                         idxs[b], sem_is[b])

    for b in range(NBUF):
        issue(b, b)

    def chunk_step(j, b):
        pltpu.make_async_copy(src_hbm.at[pl.ds(0, CH)],
                              idxs[b], sem_is[b]).wait()
        pltpu.sync_copy(ones_v, sh_c.at[idxs[b]], add=True)

        @pl.when(j + NBUF < nchunk)
        def _():
            issue(j + NBUF, b)

    def round_body(g, _):
        for b in range(NBUF):
            chunk_step(g + b, b)
        return 0

    lax.fori_loop(0, nchunk // NBUF,
                  lambda g, x: round_body(g * NBUF, x), 0, unroll=False)
    for b in range(nchunk % NBUF):
        chunk_step((nchunk // NBUF) * NBUF + b, b)

    plsc.subcore_barrier()
    pltpu.sync_copy(sh_c.at[pl.ds(r0, STRIPE)],
                    cnt_hbm.at[cid, pl.ds(r0, STRIPE)])

    @pl.when(sid == NS - 1)
    def _():
        pltpu.sync_copy(sh_c.at[pl.ds(TAIL0, TAILN)],
                        cnt_hbm.at[cid, pl.ds(TAIL0, TAILN)])


def _scnt(src):
    mesh = plsc.VectorSubcoreMesh(core_axis_name="c", subcore_axis_name="s")
    f = functools.partial(
        pl.kernel,
        mesh=mesh,
        out_type=jax.ShapeDtypeStruct((NC, N_NODES, D), jnp.float32),
        scratch_types=[
            pltpu.VMEM((CH,), jnp.int32),
            pltpu.VMEM((CH,), jnp.int32),
            pltpu.VMEM((CH, D), jnp.float32),
            pltpu.VMEM((CH, D), jnp.float32),
            pltpu.VMEM_SHARED((N_NODES, D), jnp.float32),
            pltpu.SemaphoreType.DMA,
            pltpu.SemaphoreType.DMA,
        ],
    )(_scnt_body)
    return f(src)


def _sgather_body(pa_hbm, src_hbm, r_hbm, idx0, idx1, buf0, buf1, sh_p,
                  sem_i0, sem_i1, sem_w0, sem_w1):
    cid = lax.axis_index("c")
    sid = lax.axis_index("s")
    epw = N_EDGES // (NC * NS)
    nchunk = epw // CH
    ebase = (cid * NS + sid) * epw

    idxs = (idx0, idx1)
    bufs = (buf0, buf1)
    sem_is = (sem_i0, sem_i1)
    sem_ws = (sem_w0, sem_w1)

    r0 = sid * STRIPE
    pltpu.sync_copy(pa_hbm.at[pl.ds(r0, STRIPE)], sh_p.at[pl.ds(r0, STRIPE)])

    @pl.when(sid == NS - 1)
    def _():
        pltpu.sync_copy(pa_hbm.at[pl.ds(TAIL0, TAILN)],
                        sh_p.at[pl.ds(TAIL0, TAILN)])

    plsc.subcore_barrier()

    def issue(j, b):
        pltpu.async_copy(src_hbm.at[pl.ds(ebase + j * CH, CH)],
                         idxs[b], sem_is[b])

    for b in range(NBUF):
        issue(b, b)

    def chunk_step(j, b):
        @pl.when(j >= NBUF)
        def _():
            pltpu.make_async_copy(bufs[b], r_hbm.at[pl.ds(0, CH)],
                                  sem_ws[b]).wait()
        pltpu.make_async_copy(src_hbm.at[pl.ds(0, CH)],
                              idxs[b], sem_is[b]).wait()
        pltpu.sync_copy(sh_p.at[idxs[b]], bufs[b])
        pltpu.async_copy(bufs[b], r_hbm.at[pl.ds(ebase + j * CH, CH)],
                         sem_ws[b])

        @pl.when(j + NBUF < nchunk)
        def _():
            issue(j + NBUF, b)

    def round_body(g, _):
        for b in range(NBUF):
            chunk_step(g + b, b)
        return 0

    lax.fori_loop(0, nchunk // NBUF,
                  lambda g, x: round_body(g * NBUF, x), 0, unroll=False)
    for b in range(nchunk % NBUF):
        chunk_step((nchunk // NBUF) * NBUF + b, b)
    for b in range(NBUF):
        pltpu.make_async_copy(bufs[b], r_hbm.at[pl.ds(0, CH)],
                              sem_ws[b]).wait()


def _sgather(pa, src):
    mesh = plsc.VectorSubcoreMesh(core_axis_name="c", subcore_axis_name="s")
    f = functools.partial(
        pl.kernel,
        mesh=mesh,
        out_type=jax.ShapeDtypeStruct((N_EDGES, D), jnp.float32),
        scratch_types=[
            pltpu.VMEM((CH,), jnp.int32),
            pltpu.VMEM((CH,), jnp.int32),
            pltpu.VMEM((CH, D), jnp.float32),
            pltpu.VMEM((CH, D), jnp.float32),
            pltpu.VMEM_SHARED((N_NODES, D), jnp.float32),
            pltpu.SemaphoreType.DMA,
            pltpu.SemaphoreType.DMA,
            pltpu.SemaphoreType.DMA,
            pltpu.SemaphoreType.DMA,
        ],
    )(_sgather_body)
    return f(pa, src)


def _s2_body(y_hbm, dst_hbm, out_hbm, buf0, buf1, idx0, idx1, sh_h,
             sem_i0, sem_i1, sem_d0, sem_d1):
    cid = lax.axis_index("c")
    sid = lax.axis_index("s")
    epw = N_EDGES // (NC * NS)
    nchunk = epw // CH
    ebase = (cid * NS + sid) * epw

    bufs = (buf0, buf1)
    idxs = (idx0, idx1)
    sem_is = (sem_i0, sem_i1)
    sem_ds = (sem_d0, sem_d1)

    _zero_rows(buf0, CH)
    r0 = sid * STRIPE
    _zero_spmem_stripe(sh_h, r0, STRIPE, buf0, CH)

    @pl.when(sid == NS - 1)
    def _():
        pltpu.sync_copy(buf0.at[pl.ds(0, TAILN)], sh_h.at[pl.ds(TAIL0, TAILN)])

    plsc.subcore_barrier()

    def issue(j, b):
        off = ebase + j * CH
        pltpu.async_copy(dst_hbm.at[pl.ds(off, CH)], idxs[b], sem_is[b])
        pltpu.async_copy(y_hbm.at[pl.ds(off, CH)], bufs[b], sem_ds[b])

    for b in range(NBUF):
        issue(b, b)

    def chunk_step(j, b):
        pltpu.make_async_copy(dst_hbm.at[pl.ds(0, CH)],
                              idxs[b], sem_is[b]).wait()
        pltpu.make_async_copy(y_hbm.at[pl.ds(0, CH)],
                              bufs[b], sem_ds[b]).wait()
        pltpu.sync_copy(bufs[b], sh_h.at[idxs[b]], add=True)

        @pl.when(j + NBUF < nchunk)
        def _():
            issue(j + NBUF, b)

    def round_body(g, _):
        for b in range(NBUF):
            chunk_step(g + b, b)
        return 0

    lax.fori_loop(0, nchunk // NBUF,
                  lambda g, x: round_body(g * NBUF, x), 0, unroll=False)
    for b in range(nchunk % NBUF):
        chunk_step((nchunk // NBUF) * NBUF + b, b)

    plsc.subcore_barrier()
    pltpu.sync_copy(sh_h.at[pl.ds(r0, STRIPE)],
                    out_hbm.at[cid, pl.ds(r0, STRIPE)])

    @pl.when(sid == NS - 1)
    def _():
        pltpu.sync_copy(sh_h.at[pl.ds(TAIL0, TAILN)],
                        out_hbm.at[cid, pl.ds(TAIL0, TAILN)])


def _s2(y, dst):
    mesh = plsc.VectorSubcoreMesh(core_axis_name="c", subcore_axis_name="s")
    f = functools.partial(
        pl.kernel,
        mesh=mesh,
        out_type=jax.ShapeDtypeStruct((NC, N_NODES, D), jnp.float32),
        scratch_types=[
            pltpu.VMEM((CH, D), jnp.float32),
            pltpu.VMEM((CH, D), jnp.float32),
            pltpu.VMEM((CH,), jnp.int32),
            pltpu.VMEM((CH,), jnp.int32),
            pltpu.VMEM_SHARED((N_NODES, D), jnp.float32),
            pltpu.SemaphoreType.DMA,
            pltpu.SemaphoreType.DMA,
            pltpu.SemaphoreType.DMA,
            pltpu.SemaphoreType.DMA,
        ],
    )(_s2_body)
    return f(y, dst)


NBLK = 2000


def _tadd_body(a_ref, b_ref, o_ref):
    o_ref[...] = a_ref[0] + b_ref[0]


def _tadd(parts):
    grid = (N_NODES // NBLK,)
    return pl.pallas_call(
        _tadd_body,
        grid=grid,
        in_specs=[
            pl.BlockSpec((1, NBLK, D), lambda i: (0, i, 0)),
            pl.BlockSpec((1, NBLK, D), lambda i: (1, i, 0)),
        ],
        out_specs=pl.BlockSpec((NBLK, D), lambda i: (i, 0)),
        out_shape=jax.ShapeDtypeStruct((N_NODES, D), jnp.float32),
    )(parts, parts)


def _t2_body(m_ref, w_ref, r_ref, e0_ref, y_ref):
    acc = jnp.dot(m_ref[...], w_ref[...], preferred_element_type=jnp.float32,
                  precision=jax.lax.Precision.HIGHEST)
    y_ref[...] = jnp.maximum(acc + r_ref[...] + e0_ref[...], 0.0)


def _t2(m, wbp_t, r, e0):
    grid = (N_EDGES // BLK,)
    return pl.pallas_call(
        _t2_body,
        grid=grid,
        in_specs=[
            pl.BlockSpec((BLK, D), lambda i: (i, 0)),
            pl.BlockSpec((D, D), lambda i: (0, 0)),
            pl.BlockSpec((BLK, D), lambda i: (i, 0)),
            pl.BlockSpec((1, D), lambda i: (0, 0)),
        ],
        out_specs=pl.BlockSpec((BLK, D), lambda i: (i, 0)),
        out_shape=jax.ShapeDtypeStruct((N_EDGES, D), jnp.float32),
    )(m, wbp_t, r, e0)


def _tg_body(m_ref, g_ref, s_ref):
    @pl.when(pl.program_id(0) == 0)
    def _():
        g_ref[...] = jnp.zeros_like(g_ref)
        s_ref[...] = jnp.zeros_like(s_ref)

    blk = m_ref[...]
    g_ref[...] += jnp.dot(blk.T, blk, preferred_element_type=jnp.float32,
                          precision=jax.lax.Precision.HIGHEST)
    s_ref[...] += jnp.sum(blk, axis=0, keepdims=True)


def _tg(m):
    grid = (N_EDGES // BLK,)
    return pl.pallas_call(
        _tg_body,
        grid=grid,
        in_specs=[pl.BlockSpec((BLK, D), lambda i: (i, 0))],
        out_specs=[
            pl.BlockSpec((D, D), lambda i: (0, 0)),
            pl.BlockSpec((1, D), lambda i: (0, 0)),
        ],
        out_shape=[
            jax.ShapeDtypeStruct((D, D), jnp.float32),
            jax.ShapeDtypeStruct((1, D), jnp.float32),
        ],
    )(m)


NBLK_S = 2000
NB_S = N_NODES // NBLK_S
E_F = float(N_EDGES)


def _tstats_body(hd_ref, hsrc_ref, c0_ref, c1_ref, g_ref, sm_ref, w_ref,
                 b1_ref, g1_ref, be1_ref,
                 pa_ref, e0_ref, wbp_ref,
                 sr_acc, srr_acc, x_acc, a_scr, e0_scr):
    i = pl.program_id(0)
    hp = jax.lax.Precision.HIGHEST
    wa_t = w_ref[:, :D].T
    wb = w_ref[:, D:]

    @pl.when(i == 0)
    def _():
        sr_acc[...] = jnp.zeros_like(sr_acc)
        srr_acc[...] = jnp.zeros_like(srr_acc)
        x_acc[...] = jnp.zeros_like(x_acc)

    @pl.when(i < NB_S)
    def _():
        p_blk = jnp.dot(hd_ref[...], wa_t, preferred_element_type=jnp.float32,
                        precision=hp)
        q_blk = jnp.dot(hsrc_ref[...], wb.T,
                        preferred_element_type=jnp.float32, precision=hp)
        c_blk = c0_ref[0][:, 0:1] + c1_ref[0][:, 0:1]
        sr_acc[...] += jnp.sum(c_blk * p_blk, axis=0, keepdims=True)
        srr_acc[...] += jnp.sum(c_blk * p_blk * p_blk, axis=0, keepdims=True)
        x_acc[...] += jnp.sum(p_blk * q_blk, axis=0, keepdims=True)
        pa_ref[...] = jnp.zeros_like(pa_ref)
        e0_ref[...] = jnp.zeros_like(e0_ref)
        wbp_ref[...] = jnp.zeros_like(wbp_ref)

    @pl.when(i == NB_S)
    def _():
        b1 = b1_ref[...]
        colsum_q = jnp.dot(sm_ref[...], wb.T,
                           preferred_element_type=jnp.float32, precision=hp)
        wbg = jnp.dot(wb, g_ref[...], preferred_element_type=jnp.float32,
                      precision=hp)
        colsumsq_q = jnp.sum(wbg * wb, axis=1, keepdims=True).T
        sum_z = colsum_q + sr_acc[...] + E_F * b1
        sumsq_z = (colsumsq_q + srr_acc[...] + E_F * b1 * b1
                   + 2.0 * x_acc[...] + 2.0 * b1 * colsum_q
                   + 2.0 * b1 * sr_acc[...])
        mean = sum_z / E_F
        var = sumsq_z / E_F - mean * mean
        a = g1_ref[...] / jnp.sqrt(var + 1e-5)
        a_scr[...] = a
        e0_scr[...] = a * (b1 - mean) + be1_ref[...]

    @pl.when(i >= NB_S)
    def _():
        a = a_scr[...]
        p_blk = jnp.dot(hd_ref[...], wa_t, preferred_element_type=jnp.float32,
                        precision=hp)
        pa_ref[...] = p_blk * a
        e0_ref[...] = e0_scr[...]
        wbp_ref[...] = wb.T * a


def _tstats(h_dst, h_src, cnt, G, sm, W1, b1, g1, be1):
    grid = (2 * NB_S,)
    bm = lambda i: (i % NB_S, 0)
    zero2 = lambda i: (0, 0)
    return pl.pallas_call(
        _tstats_body,
        grid=grid,
        in_specs=[
            pl.BlockSpec((NBLK_S, D), bm),
            pl.BlockSpec((NBLK_S, D), bm),
            pl.BlockSpec((1, NBLK_S, D), lambda i: (0, i % NB_S, 0)),
            pl.BlockSpec((1, NBLK_S, D), lambda i: (1, i % NB_S, 0)),
            pl.BlockSpec((D, D), zero2),
            pl.BlockSpec((1, D), zero2),
            pl.BlockSpec((D, 2 * D), zero2),
            pl.BlockSpec((1, D), zero2),
            pl.BlockSpec((1, D), zero2),
            pl.BlockSpec((1, D), zero2),
        ],
        out_specs=[
            pl.BlockSpec((NBLK_S, D), bm),
            pl.BlockSpec((1, D), zero2),
            pl.BlockSpec((D, D), zero2),
        ],
        out_shape=[
            jax.ShapeDtypeStruct((N_NODES, D), jnp.float32),
            jax.ShapeDtypeStruct((1, D), jnp.float32),
            jax.ShapeDtypeStruct((D, D), jnp.float32),
        ],
        scratch_shapes=[
            pltpu.VMEM((1, D), jnp.float32),
            pltpu.VMEM((1, D), jnp.float32),
            pltpu.VMEM((1, D), jnp.float32),
            pltpu.VMEM((1, D), jnp.float32),
            pltpu.VMEM((1, D), jnp.float32),
        ],
    )(h_dst, h_src, cnt, cnt, G, sm, W1,
      b1[None, :], g1[None, :], be1[None, :])


def kernel(m, edge_index, W1, b1, g1, be1, W2, b2, g2, be2):
    ei = edge_index.astype(jnp.int32)
    src = ei[0]
    dst = ei[1]

    hh = _s1(m, ei)
    cnt = _scnt(src)
    G, sm = _tg(m)
    pa, e0, wbp_t = _tstats(hh[0], hh[1], cnt, G, sm, W1, b1, g1, be1)
    r = _sgather(pa, src)
    y = _t2(m, wbp_t, r, e0)
    parts = _s2(y, dst)
    h2 = _tadd(parts)
    return h2


# speedup vs baseline: 5.0222x; 1.0383x over previous
import functools
import jax
import jax.numpy as jnp
from jax import lax
from jax.experimental import pallas as pl
from jax.experimental.pallas import tpu as pltpu
from jax.experimental.pallas import tpu_sc as plsc

N_NODES = 10000
N_EDGES = 320000
D = 128
BLK = 2560
assert N_EDGES % BLK == 0

NC = 2
NS = 16
CH = 80
NBUF = 2

STRIPE = 624
TAIL0 = NS * STRIPE
TAILN = N_NODES - TAIL0


def _zero_rows(buf, nrows):
    zv = jnp.zeros((16,), jnp.float32)

    def body(i, _):
        for c in range(D // 16):
            buf[i, pl.ds(c * 16, 16)] = zv
        return 0

    lax.fori_loop(0, nrows, body, 0)


def _zero_spmem_stripe(sh, row0, nrows, zbuf, zrows):
    full, rem = nrows // zrows, nrows % zrows
    for k in range(full):
        pltpu.sync_copy(zbuf.at[pl.ds(0, zrows)],
                        sh.at[pl.ds(row0 + k * zrows, zrows)])
    if rem:
        pltpu.sync_copy(zbuf.at[pl.ds(0, rem)],
                        sh.at[pl.ds(row0 + full * zrows, rem)])


NBUF1 = 4
LA1 = 2


def _s1_body(m_hbm, dst_hbm, src_hbm, out_hbm, *rest):
    bufs = rest[0:NBUF1]
    idxs = rest[NBUF1:2 * NBUF1]
    sh_h = rest[2 * NBUF1]
    sem_is = rest[2 * NBUF1 + 1:2 * NBUF1 + 1 + NBUF1]
    sem_ds = rest[2 * NBUF1 + 1 + NBUF1:2 * NBUF1 + 1 + 2 * NBUF1]
    sem_ss = rest[2 * NBUF1 + 1 + 2 * NBUF1:2 * NBUF1 + 1 + 3 * NBUF1]

    cid = lax.axis_index("c")
    sid = lax.axis_index("s")
    epw = N_EDGES // NS
    nchunk = epw // CH
    ebase = sid * epw

    _zero_rows(bufs[0], CH)
    r0 = sid * STRIPE
    _zero_spmem_stripe(sh_h, r0, STRIPE, bufs[0], CH)

    @pl.when(sid == NS - 1)
    def _():
        pltpu.sync_copy(bufs[0].at[pl.ds(0, TAILN)],
                        sh_h.at[pl.ds(TAIL0, TAILN)])

    plsc.subcore_barrier()

    def run_pipeline(idx_hbm):
        def issue(j, b):
            off = ebase + j * CH
            pltpu.async_copy(idx_hbm.at[pl.ds(off, CH)], idxs[b], sem_is[b])
            pltpu.async_copy(m_hbm.at[pl.ds(off, CH)], bufs[b], sem_ds[b])

        for b in range(LA1):
            issue(b, b)

        def chunk_step(j, b):
            bb = (b + LA1) % NBUF1

            @pl.when(j + LA1 < nchunk)
            def _():
                @pl.when(j >= NBUF1 - LA1)
                def _():
                    pltpu.make_async_copy(bufs[bb], sh_h.at[idxs[bb]],
                                          sem_ss[bb]).wait()
                issue(j + LA1, bb)

            pltpu.make_async_copy(idx_hbm.at[pl.ds(0, CH)],
                                  idxs[b], sem_is[b]).wait()
            pltpu.make_async_copy(m_hbm.at[pl.ds(0, CH)],
                                  bufs[b], sem_ds[b]).wait()
            pltpu.async_copy(bufs[b], sh_h.at[idxs[b]], sem_ss[b], add=True)

        def round_body(g, _):
            for b in range(NBUF1):
                chunk_step(g + b, b)
            return 0

        lax.fori_loop(0, nchunk // NBUF1,
                      lambda g, x: round_body(g * NBUF1, x), 0, unroll=False)
        for k in range(nchunk % NBUF1):
            j = (nchunk // NBUF1) * NBUF1 + k
            chunk_step(j, j % NBUF1)
        for b in range(NBUF1):
            pltpu.make_async_copy(bufs[b], sh_h.at[idxs[b]], sem_ss[b]).wait()

    @pl.when(cid == 0)
    def _():
        run_pipeline(dst_hbm)

    @pl.when(cid == 1)
    def _():
        run_pipeline(src_hbm)

    plsc.subcore_barrier()

    pltpu.sync_copy(sh_h.at[pl.ds(r0, STRIPE)],
                    out_hbm.at[cid, pl.ds(r0, STRIPE)])

    @pl.when(sid == NS - 1)
    def _():
        pltpu.sync_copy(sh_h.at[pl.ds(TAIL0, TAILN)],
                        out_hbm.at[cid, pl.ds(TAIL0, TAILN)])


def _s1(m, ei):
    mesh = plsc.VectorSubcoreMesh(core_axis_name="c", subcore_axis_name="s")
    f = functools.partial(
        pl.kernel,
        mesh=mesh,
        out_type=jax.ShapeDtypeStruct((NC, N_NODES, D), jnp.float32),
        scratch_types=(
            [pltpu.VMEM((CH, D), jnp.float32)] * NBUF1
            + [pltpu.VMEM((CH,), jnp.int32)] * NBUF1
            + [pltpu.VMEM_SHARED((N_NODES, D), jnp.float32)]
            + [pltpu.SemaphoreType.DMA] * (3 * NBUF1)
        ),
    )(_s1_body)
    return f(m, ei[1], ei[0])


def _scnt_body(src_hbm, cnt_hbm, idx0, idx1, ones_v, zc_v, sh_c,
               sem_i0, sem_i1):
    cid = lax.axis_index("c")
    sid = lax.axis_index("s")
    epw = N_EDGES // (NC * NS)
    nchunk = epw // CH
    ebase = (cid * NS + sid) * epw

    idxs = (idx0, idx1)
    sem_is = (sem_i0, sem_i1)

    ov = jnp.ones((16,), jnp.float32)

    def fill(i, _):
        for cc in range(D // 16):
            ones_v[i, pl.ds(cc * 16, 16)] = ov
        return 0
    lax.fori_loop(0, CH, fill, 0)
    _zero_rows(zc_v, CH)

    r0 = sid * STRIPE
    _zero_spmem_stripe(sh_c, r0, STRIPE, zc_v, CH)

    @pl.when(sid == NS - 1)
    def _():
        pltpu.sync_copy(zc_v.at[pl.ds(0, TAILN)], sh_c.at[pl.ds(TAIL0, TAILN)])

    plsc.subcore_barrier()

    def issue(j, b):
        pltpu.async_copy(src_hbm.at[pl.ds(ebase + j * CH, CH)],
                         idxs[b], sem_is[b])

    for b in range(NBUF):
        issue(b, b)

    def chunk_step(j, b):
        pltpu.make_async_copy(src_hbm.at[pl.ds(0, CH)],
                              idxs[b], sem_is[b]).wait()
        pltpu.sync_copy(ones_v, sh_c.at[idxs[b]], add=True)

        @pl.when(j + NBUF < nchunk)
        def _():
            issue(j + NBUF, b)

    def round_body(g, _):
        for b in range(NBUF):
            chunk_step(g + b, b)
        return 0

    lax.fori_loop(0, nchunk // NBUF,
                  lambda g, x: round_body(g * NBUF, x), 0, unroll=False)
    for b in range(nchunk % NBUF):
        chunk_step((nchunk // NBUF) * NBUF + b, b)

    plsc.subcore_barrier()
    pltpu.sync_copy(sh_c.at[pl.ds(r0, STRIPE)],
                    cnt_hbm.at[cid, pl.ds(r0, STRIPE)])

    @pl.when(sid == NS - 1)
    def _():
        pltpu.sync_copy(sh_c.at[pl.ds(TAIL0, TAILN)],
                        cnt_hbm.at[cid, pl.ds(TAIL0, TAILN)])


def _scnt(src):
    mesh = plsc.VectorSubcoreMesh(core_axis_name="c", subcore_axis_name="s")
    f = functools.partial(
        pl.kernel,
        mesh=mesh,
        out_type=jax.ShapeDtypeStruct((NC, N_NODES, D), jnp.float32),
        scratch_types=[
            pltpu.VMEM((CH,), jnp.int32),
            pltpu.VMEM((CH,), jnp.int32),
            pltpu.VMEM((CH, D), jnp.float32),
            pltpu.VMEM((CH, D), jnp.float32),
            pltpu.VMEM_SHARED((N_NODES, D), jnp.float32),
            pltpu.SemaphoreType.DMA,
            pltpu.SemaphoreType.DMA,
        ],
    )(_scnt_body)
    return f(src)


def _sgather_body(pa_hbm, src_hbm, r_hbm, idx0, idx1, buf0, buf1, sh_p,
                  sem_i0, sem_i1, sem_w0, sem_w1):
    cid = lax.axis_index("c")
    sid = lax.axis_index("s")
    epw = N_EDGES // (NC * NS)
    nchunk = epw // CH
    ebase = (cid * NS + sid) * epw

    idxs = (idx0, idx1)
    bufs = (buf0, buf1)
    sem_is = (sem_i0, sem_i1)
    sem_ws = (sem_w0, sem_w1)

    r0 = sid * STRIPE
    pltpu.sync_copy(pa_hbm.at[pl.ds(r0, STRIPE)], sh_p.at[pl.ds(r0, STRIPE)])

    @pl.when(sid == NS - 1)
    def _():
        pltpu.sync_copy(pa_hbm.at[pl.ds(TAIL0, TAILN)],
                        sh_p.at[pl.ds(TAIL0, TAILN)])

    plsc.subcore_barrier()

    def issue(j, b):
        pltpu.async_copy(src_hbm.at[pl.ds(ebase + j * CH, CH)],
                         idxs[b], sem_is[b])

    for b in range(NBUF):
        issue(b, b)

    def chunk_step(j, b):
        @pl.when(j >= NBUF)
        def _():
            pltpu.make_async_copy(bufs[b], r_hbm.at[pl.ds(0, CH)],
                                  sem_ws[b]).wait()
        pltpu.make_async_copy(src_hbm.at[pl.ds(0, CH)],
                              idxs[b], sem_is[b]).wait()
        pltpu.sync_copy(sh_p.at[idxs[b]], bufs[b])
        pltpu.async_copy(bufs[b], r_hbm.at[pl.ds(ebase + j * CH, CH)],
                         sem_ws[b])

        @pl.when(j + NBUF < nchunk)
        def _():
            issue(j + NBUF, b)

    def round_body(g, _):
        for b in range(NBUF):
            chunk_step(g + b, b)
        return 0

    lax.fori_loop(0, nchunk // NBUF,
                  lambda g, x: round_body(g * NBUF, x), 0, unroll=False)
    for b in range(nchunk % NBUF):
        chunk_step((nchunk // NBUF) * NBUF + b, b)
    for b in range(NBUF):
        pltpu.make_async_copy(bufs[b], r_hbm.at[pl.ds(0, CH)],
                              sem_ws[b]).wait()


def _sgather(pa, src):
    mesh = plsc.VectorSubcoreMesh(core_axis_name="c", subcore_axis_name="s")
    f = functools.partial(
        pl.kernel,
        mesh=mesh,
        out_type=jax.ShapeDtypeStruct((N_EDGES, D), jnp.float32),
        scratch_types=[
            pltpu.VMEM((CH,), jnp.int32),
            pltpu.VMEM((CH,), jnp.int32),
            pltpu.VMEM((CH, D), jnp.float32),
            pltpu.VMEM((CH, D), jnp.float32),
            pltpu.VMEM_SHARED((N_NODES, D), jnp.float32),
            pltpu.SemaphoreType.DMA,
            pltpu.SemaphoreType.DMA,
            pltpu.SemaphoreType.DMA,
            pltpu.SemaphoreType.DMA,
        ],
    )(_sgather_body)
    return f(pa, src)


def _s2_body(y_hbm, dst_hbm, out_hbm, buf0, buf1, idx0, idx1, sh_h,
             sem_i0, sem_i1, sem_d0, sem_d1):
    cid = lax.axis_index("c")
    sid = lax.axis_index("s")
    epw = N_EDGES // (NC * NS)
    nchunk = epw // CH
    ebase = (cid * NS + sid) * epw

    bufs = (buf0, buf1)
    idxs = (idx0, idx1)
    sem_is = (sem_i0, sem_i1)
    sem_ds = (sem_d0, sem_d1)

    _zero_rows(buf0, CH)
    r0 = sid * STRIPE
    _zero_spmem_stripe(sh_h, r0, STRIPE, buf0, CH)

    @pl.when(sid == NS - 1)
    def _():
        pltpu.sync_copy(buf0.at[pl.ds(0, TAILN)], sh_h.at[pl.ds(TAIL0, TAILN)])

    plsc.subcore_barrier()

    def issue(j, b):
        off = ebase + j * CH
        pltpu.async_copy(dst_hbm.at[pl.ds(off, CH)], idxs[b], sem_is[b])
        pltpu.async_copy(y_hbm.at[pl.ds(off, CH)], bufs[b], sem_ds[b])

    for b in range(NBUF):
        issue(b, b)

    def chunk_step(j, b):
        pltpu.make_async_copy(dst_hbm.at[pl.ds(0, CH)],
                              idxs[b], sem_is[b]).wait()
        pltpu.make_async_copy(y_hbm.at[pl.ds(0, CH)],
                              bufs[b], sem_ds[b]).wait()
        pltpu.sync_copy(bufs[b], sh_h.at[idxs[b]], add=True)

        @pl.when(j + NBUF < nchunk)
        def _():
            issue(j + NBUF, b)

    def round_body(g, _):
        for b in range(NBUF):
            chunk_step(g + b, b)
        return 0

    lax.fori_loop(0, nchunk // NBUF,
                  lambda g, x: round_body(g * NBUF, x), 0, unroll=False)
    for b in range(nchunk % NBUF):
        chunk_step((nchunk // NBUF) * NBUF + b, b)

    plsc.subcore_barrier()
    pltpu.sync_copy(sh_h.at[pl.ds(r0, STRIPE)],
                    out_hbm.at[cid, pl.ds(r0, STRIPE)])

    @pl.when(sid == NS - 1)
    def _():
        pltpu.sync_copy(sh_h.at[pl.ds(TAIL0, TAILN)],
                        out_hbm.at[cid, pl.ds(TAIL0, TAILN)])


def _s2(y, dst):
    mesh = plsc.VectorSubcoreMesh(core_axis_name="c", subcore_axis_name="s")
    f = functools.partial(
        pl.kernel,
        mesh=mesh,
        out_type=jax.ShapeDtypeStruct((NC, N_NODES, D), jnp.float32),
        scratch_types=[
            pltpu.VMEM((CH, D), jnp.float32),
            pltpu.VMEM((CH, D), jnp.float32),
            pltpu.VMEM((CH,), jnp.int32),
            pltpu.VMEM((CH,), jnp.int32),
            pltpu.VMEM_SHARED((N_NODES, D), jnp.float32),
            pltpu.SemaphoreType.DMA,
            pltpu.SemaphoreType.DMA,
            pltpu.SemaphoreType.DMA,
            pltpu.SemaphoreType.DMA,
        ],
    )(_s2_body)
    return f(y, dst)


NBLK = 2000


def _tadd_body(a_ref, b_ref, o_ref):
    o_ref[...] = a_ref[0] + b_ref[0]


def _tadd(parts):
    grid = (N_NODES // NBLK,)
    return pl.pallas_call(
        _tadd_body,
        grid=grid,
        in_specs=[
            pl.BlockSpec((1, NBLK, D), lambda i: (0, i, 0)),
            pl.BlockSpec((1, NBLK, D), lambda i: (1, i, 0)),
        ],
        out_specs=pl.BlockSpec((NBLK, D), lambda i: (i, 0)),
        out_shape=jax.ShapeDtypeStruct((N_NODES, D), jnp.float32),
    )(parts, parts)


def _t2_body(m_ref, w_ref, r_ref, e0_ref, y_ref):
    acc = jnp.dot(m_ref[...], w_ref[...], preferred_element_type=jnp.float32,
                  precision=jax.lax.Precision.HIGHEST)
    y_ref[...] = jnp.maximum(acc + r_ref[...] + e0_ref[...], 0.0)


def _t2(m, wbp_t, r, e0):
    grid = (N_EDGES // BLK,)
    return pl.pallas_call(
        _t2_body,
        grid=grid,
        in_specs=[
            pl.BlockSpec((BLK, D), lambda i: (i, 0)),
            pl.BlockSpec((D, D), lambda i: (0, 0)),
            pl.BlockSpec((BLK, D), lambda i: (i, 0)),
            pl.BlockSpec((1, D), lambda i: (0, 0)),
        ],
        out_specs=pl.BlockSpec((BLK, D), lambda i: (i, 0)),
        out_shape=jax.ShapeDtypeStruct((N_EDGES, D), jnp.float32),
    )(m, wbp_t, r, e0)


def _tg_body(m_ref, g_ref, s_ref):
    @pl.when(pl.program_id(0) == 0)
    def _():
        g_ref[...] = jnp.zeros_like(g_ref)
        s_ref[...] = jnp.zeros_like(s_ref)

    blk = m_ref[...]
    g_ref[...] += jnp.dot(blk.T, blk, preferred_element_type=jnp.float32,
                          precision=jax.lax.Precision.HIGHEST)
    s_ref[...] += jnp.sum(blk, axis=0, keepdims=True)


def _tg(m):
    grid = (N_EDGES // BLK,)
    return pl.pallas_call(
        _tg_body,
        grid=grid,
        in_specs=[pl.BlockSpec((BLK, D), lambda i: (i, 0))],
        out_specs=[
            pl.BlockSpec((D, D), lambda i: (0, 0)),
            pl.BlockSpec((1, D), lambda i: (0, 0)),
        ],
        out_shape=[
            jax.ShapeDtypeStruct((D, D), jnp.float32),
            jax.ShapeDtypeStruct((1, D), jnp.float32),
        ],
    )(m)


NBLK_S = 2000
NB_S = N_NODES // NBLK_S
E_F = float(N_EDGES)


def _tstats_body(hd_ref, hsrc_ref, c0_ref, c1_ref, g_ref, sm_ref, w_ref,
                 b1_ref, g1_ref, be1_ref,
                 pa_ref, e0_ref, wbp_ref,
                 sr_acc, srr_acc, x_acc, a_scr, e0_scr):
    i = pl.program_id(0)
    hp = jax.lax.Precision.HIGHEST
    wa_t = w_ref[:, :D].T
    wb = w_ref[:, D:]

    @pl.when(i == 0)
    def _():
        sr_acc[...] = jnp.zeros_like(sr_acc)
        srr_acc[...] = jnp.zeros_like(srr_acc)
        x_acc[...] = jnp.zeros_like(x_acc)

    @pl.when(i < NB_S)
    def _():
        p_blk = jnp.dot(hd_ref[...], wa_t, preferred_element_type=jnp.float32,
                        precision=hp)
        q_blk = jnp.dot(hsrc_ref[...], wb.T,
                        preferred_element_type=jnp.float32, precision=hp)
        c_blk = c0_ref[0][:, 0:1] + c1_ref[0][:, 0:1]
        sr_acc[...] += jnp.sum(c_blk * p_blk, axis=0, keepdims=True)
        srr_acc[...] += jnp.sum(c_blk * p_blk * p_blk, axis=0, keepdims=True)
        x_acc[...] += jnp.sum(p_blk * q_blk, axis=0, keepdims=True)
        pa_ref[...] = jnp.zeros_like(pa_ref)
        e0_ref[...] = jnp.zeros_like(e0_ref)
        wbp_ref[...] = jnp.zeros_like(wbp_ref)

    @pl.when(i == NB_S)
    def _():
        b1 = b1_ref[...]
        colsum_q = jnp.dot(sm_ref[...], wb.T,
                           preferred_element_type=jnp.float32, precision=hp)
        wbg = jnp.dot(wb, g_ref[...], preferred_element_type=jnp.float32,
                      precision=hp)
        colsumsq_q = jnp.sum(wbg * wb, axis=1, keepdims=True).T
        sum_z = colsum_q + sr_acc[...] + E_F * b1
        sumsq_z = (colsumsq_q + srr_acc[...] + E_F * b1 * b1
                   + 2.0 * x_acc[...] + 2.0 * b1 * colsum_q
                   + 2.0 * b1 * sr_acc[...])
        mean = sum_z / E_F
        var = sumsq_z / E_F - mean * mean
        a = g1_ref[...] / jnp.sqrt(var + 1e-5)
        a_scr[...] = a
        e0_scr[...] = a * (b1 - mean) + be1_ref[...]

    @pl.when(i >= NB_S)
    def _():
        a = a_scr[...]
        p_blk = jnp.dot(hd_ref[...], wa_t, preferred_element_type=jnp.float32,
                        precision=hp)
        pa_ref[...] = p_blk * a
        e0_ref[...] = e0_scr[...]
        wbp_ref[...] = wb.T * a


def _tstats(h_dst, h_src, cnt, G, sm, W1, b1, g1, be1):
    grid = (2 * NB_S,)
    bm = lambda i: (i % NB_S, 0)
    zero2 = lambda i: (0, 0)
    return pl.pallas_call(
        _tstats_body,
        grid=grid,
        in_specs=[
            pl.BlockSpec((NBLK_S, D), bm),
            pl.BlockSpec((NBLK_S, D), bm),
            pl.BlockSpec((1, NBLK_S, D), lambda i: (0, i % NB_S, 0)),
            pl.BlockSpec((1, NBLK_S, D), lambda i: (1, i % NB_S, 0)),
            pl.BlockSpec((D, D), zero2),
            pl.BlockSpec((1, D), zero2),
            pl.BlockSpec((D, 2 * D), zero2),
            pl.BlockSpec((1, D), zero2),
            pl.BlockSpec((1, D), zero2),
            pl.BlockSpec((1, D), zero2),
        ],
        out_specs=[
            pl.BlockSpec((NBLK_S, D), bm),
            pl.BlockSpec((1, D), zero2),
            pl.BlockSpec((D, D), zero2),
        ],
        out_shape=[
            jax.ShapeDtypeStruct((N_NODES, D), jnp.float32),
            jax.ShapeDtypeStruct((1, D), jnp.float32),
            jax.ShapeDtypeStruct((D, D), jnp.float32),
        ],
        scratch_shapes=[
            pltpu.VMEM((1, D), jnp.float32),
            pltpu.VMEM((1, D), jnp.float32),
            pltpu.VMEM((1, D), jnp.float32),
            pltpu.VMEM((1, D), jnp.float32),
            pltpu.VMEM((1, D), jnp.float32),
        ],
    )(h_dst, h_src, cnt, cnt, G, sm, W1,
      b1[None, :], g1[None, :], be1[None, :])


def kernel(m, edge_index, W1, b1, g1, be1, W2, b2, g2, be2):
    ei = edge_index.astype(jnp.int32)
    src = ei[0]
    dst = ei[1]

    hh = _s1(m, ei)
    cnt = _scnt(src)
    G, sm = _tg(m)
    pa, e0, wbp_t = _tstats(hh[0], hh[1], cnt, G, sm, W1, b1, g1, be1)
    r = _sgather(pa, src)
    y = _t2(m, wbp_t, r, e0)
    parts = _s2(y, dst)
    h2 = _tadd(parts)
    return h2


# speedup vs baseline: 5.1828x; 1.0320x over previous
import functools
import jax
import jax.numpy as jnp
from jax import lax
from jax.experimental import pallas as pl
from jax.experimental.pallas import tpu as pltpu
from jax.experimental.pallas import tpu_sc as plsc

N_NODES = 10000
N_EDGES = 320000
D = 128
BLK = 2560
assert N_EDGES % BLK == 0

NC = 2
NS = 16
CH = 80
NBUF = 2

STRIPE = 624
TAIL0 = NS * STRIPE
TAILN = N_NODES - TAIL0


def _zero_rows(buf, nrows):
    zv = jnp.zeros((16,), jnp.float32)

    def body(i, _):
        for c in range(D // 16):
            buf[i, pl.ds(c * 16, 16)] = zv
        return 0

    lax.fori_loop(0, nrows, body, 0)


def _zero_spmem_stripe(sh, row0, nrows, zbuf, zrows):
    full, rem = nrows // zrows, nrows % zrows
    for k in range(full):
        pltpu.sync_copy(zbuf.at[pl.ds(0, zrows)],
                        sh.at[pl.ds(row0 + k * zrows, zrows)])
    if rem:
        pltpu.sync_copy(zbuf.at[pl.ds(0, rem)],
                        sh.at[pl.ds(row0 + full * zrows, rem)])


NBUF1 = 4
LA1 = 2


def _s1_body(m_hbm, dst_hbm, src_hbm, out_hbm, *rest):
    bufs = rest[0:NBUF1]
    idxs = rest[NBUF1:2 * NBUF1]
    sh_h = rest[2 * NBUF1]
    sem_is = rest[2 * NBUF1 + 1:2 * NBUF1 + 1 + NBUF1]
    sem_ds = rest[2 * NBUF1 + 1 + NBUF1:2 * NBUF1 + 1 + 2 * NBUF1]
    sem_ss = rest[2 * NBUF1 + 1 + 2 * NBUF1:2 * NBUF1 + 1 + 3 * NBUF1]

    cid = lax.axis_index("c")
    sid = lax.axis_index("s")
    epw = N_EDGES // NS
    nchunk = epw // CH
    ebase = sid * epw

    _zero_rows(bufs[0], CH)
    r0 = sid * STRIPE
    _zero_spmem_stripe(sh_h, r0, STRIPE, bufs[0], CH)

    @pl.when(sid == NS - 1)
    def _():
        pltpu.sync_copy(bufs[0].at[pl.ds(0, TAILN)],
                        sh_h.at[pl.ds(TAIL0, TAILN)])

    plsc.subcore_barrier()

    def run_pipeline(idx_hbm):
        def issue(j, b):
            off = ebase + j * CH
            pltpu.async_copy(idx_hbm.at[pl.ds(off, CH)], idxs[b], sem_is[b])
            pltpu.async_copy(m_hbm.at[pl.ds(off, CH)], bufs[b], sem_ds[b])

        for b in range(LA1):
            issue(b, b)

        def chunk_step(j, b):
            bb = (b + LA1) % NBUF1

            @pl.when(j + LA1 < nchunk)
            def _():
                @pl.when(j >= NBUF1 - LA1)
                def _():
                    pltpu.make_async_copy(bufs[bb], sh_h.at[idxs[bb]],
                                          sem_ss[bb]).wait()
                issue(j + LA1, bb)

            pltpu.make_async_copy(idx_hbm.at[pl.ds(0, CH)],
                                  idxs[b], sem_is[b]).wait()
            pltpu.make_async_copy(m_hbm.at[pl.ds(0, CH)],
                                  bufs[b], sem_ds[b]).wait()
            pltpu.async_copy(bufs[b], sh_h.at[idxs[b]], sem_ss[b], add=True)

        def round_body(g, _):
            for b in range(NBUF1):
                chunk_step(g + b, b)
            return 0

        lax.fori_loop(0, nchunk // NBUF1,
                      lambda g, x: round_body(g * NBUF1, x), 0, unroll=False)
        for k in range(nchunk % NBUF1):
            j = (nchunk // NBUF1) * NBUF1 + k
            chunk_step(j, j % NBUF1)
        for b in range(NBUF1):
            pltpu.make_async_copy(bufs[b], sh_h.at[idxs[b]], sem_ss[b]).wait()

    @pl.when(cid == 0)
    def _():
        run_pipeline(dst_hbm)

    @pl.when(cid == 1)
    def _():
        run_pipeline(src_hbm)

    plsc.subcore_barrier()

    pltpu.sync_copy(sh_h.at[pl.ds(r0, STRIPE)],
                    out_hbm.at[cid, pl.ds(r0, STRIPE)])

    @pl.when(sid == NS - 1)
    def _():
        pltpu.sync_copy(sh_h.at[pl.ds(TAIL0, TAILN)],
                        out_hbm.at[cid, pl.ds(TAIL0, TAILN)])


def _s1(m, ei):
    mesh = plsc.VectorSubcoreMesh(core_axis_name="c", subcore_axis_name="s")
    f = functools.partial(
        pl.kernel,
        mesh=mesh,
        out_type=jax.ShapeDtypeStruct((NC, N_NODES, D), jnp.float32),
        scratch_types=(
            [pltpu.VMEM((CH, D), jnp.float32)] * NBUF1
            + [pltpu.VMEM((CH,), jnp.int32)] * NBUF1
            + [pltpu.VMEM_SHARED((N_NODES, D), jnp.float32)]
            + [pltpu.SemaphoreType.DMA] * (3 * NBUF1)
        ),
    )(_s1_body)
    return f(m, ei[1], ei[0])


def _scnt_body(src_hbm, cnt_hbm, idx0, idx1, ones_v, zc_v, sh_c,
               sem_i0, sem_i1):
    cid = lax.axis_index("c")
    sid = lax.axis_index("s")
    epw = N_EDGES // (NC * NS)
    nchunk = epw // CH
    ebase = (cid * NS + sid) * epw

    idxs = (idx0, idx1)
    sem_is = (sem_i0, sem_i1)

    ov = jnp.ones((16,), jnp.float32)

    def fill(i, _):
        for cc in range(D // 16):
            ones_v[i, pl.ds(cc * 16, 16)] = ov
        return 0
    lax.fori_loop(0, CH, fill, 0)
    _zero_rows(zc_v, CH)

    r0 = sid * STRIPE
    _zero_spmem_stripe(sh_c, r0, STRIPE, zc_v, CH)

    @pl.when(sid == NS - 1)
    def _():
        pltpu.sync_copy(zc_v.at[pl.ds(0, TAILN)], sh_c.at[pl.ds(TAIL0, TAILN)])

    plsc.subcore_barrier()

    def issue(j, b):
        pltpu.async_copy(src_hbm.at[pl.ds(ebase + j * CH, CH)],
                         idxs[b], sem_is[b])

    for b in range(NBUF):
        issue(b, b)

    def chunk_step(j, b):
        pltpu.make_async_copy(src_hbm.at[pl.ds(0, CH)],
                              idxs[b], sem_is[b]).wait()
        pltpu.sync_copy(ones_v, sh_c.at[idxs[b]], add=True)

        @pl.when(j + NBUF < nchunk)
        def _():
            issue(j + NBUF, b)

    def round_body(g, _):
        for b in range(NBUF):
            chunk_step(g + b, b)
        return 0

    lax.fori_loop(0, nchunk // NBUF,
                  lambda g, x: round_body(g * NBUF, x), 0, unroll=False)
    for b in range(nchunk % NBUF):
        chunk_step((nchunk // NBUF) * NBUF + b, b)

    plsc.subcore_barrier()
    pltpu.sync_copy(sh_c.at[pl.ds(r0, STRIPE)],
                    cnt_hbm.at[cid, pl.ds(r0, STRIPE)])

    @pl.when(sid == NS - 1)
    def _():
        pltpu.sync_copy(sh_c.at[pl.ds(TAIL0, TAILN)],
                        cnt_hbm.at[cid, pl.ds(TAIL0, TAILN)])


def _scnt(src):
    mesh = plsc.VectorSubcoreMesh(core_axis_name="c", subcore_axis_name="s")
    f = functools.partial(
        pl.kernel,
        mesh=mesh,
        out_type=jax.ShapeDtypeStruct((NC, N_NODES, D), jnp.float32),
        scratch_types=[
            pltpu.VMEM((CH,), jnp.int32),
            pltpu.VMEM((CH,), jnp.int32),
            pltpu.VMEM((CH, D), jnp.float32),
            pltpu.VMEM((CH, D), jnp.float32),
            pltpu.VMEM_SHARED((N_NODES, D), jnp.float32),
            pltpu.SemaphoreType.DMA,
            pltpu.SemaphoreType.DMA,
        ],
    )(_scnt_body)
    return f(src)


def _sgather_body(pa_hbm, src_hbm, r_hbm, idx0, idx1, buf0, buf1, sh_p,
                  sem_i0, sem_i1, sem_w0, sem_w1):
    cid = lax.axis_index("c")
    sid = lax.axis_index("s")
    epw = N_EDGES // (NC * NS)
    nchunk = epw // CH
    ebase = (cid * NS + sid) * epw

    idxs = (idx0, idx1)
    bufs = (buf0, buf1)
    sem_is = (sem_i0, sem_i1)
    sem_ws = (sem_w0, sem_w1)

    r0 = sid * STRIPE
    pltpu.sync_copy(pa_hbm.at[pl.ds(r0, STRIPE)], sh_p.at[pl.ds(r0, STRIPE)])

    @pl.when(sid == NS - 1)
    def _():
        pltpu.sync_copy(pa_hbm.at[pl.ds(TAIL0, TAILN)],
                        sh_p.at[pl.ds(TAIL0, TAILN)])

    plsc.subcore_barrier()

    def issue(j, b):
        pltpu.async_copy(src_hbm.at[pl.ds(ebase + j * CH, CH)],
                         idxs[b], sem_is[b])

    for b in range(NBUF):
        issue(b, b)

    def chunk_step(j, b):
        @pl.when(j >= NBUF)
        def _():
            pltpu.make_async_copy(bufs[b], r_hbm.at[pl.ds(0, CH)],
                                  sem_ws[b]).wait()
        pltpu.make_async_copy(src_hbm.at[pl.ds(0, CH)],
                              idxs[b], sem_is[b]).wait()
        pltpu.sync_copy(sh_p.at[idxs[b]], bufs[b])
        pltpu.async_copy(bufs[b], r_hbm.at[pl.ds(ebase + j * CH, CH)],
                         sem_ws[b])

        @pl.when(j + NBUF < nchunk)
        def _():
            issue(j + NBUF, b)

    def round_body(g, _):
        for b in range(NBUF):
            chunk_step(g + b, b)
        return 0

    lax.fori_loop(0, nchunk // NBUF,
                  lambda g, x: round_body(g * NBUF, x), 0, unroll=False)
    for b in range(nchunk % NBUF):
        chunk_step((nchunk // NBUF) * NBUF + b, b)
    for b in range(NBUF):
        pltpu.make_async_copy(bufs[b], r_hbm.at[pl.ds(0, CH)],
                              sem_ws[b]).wait()


def _sgather(pa, src):
    mesh = plsc.VectorSubcoreMesh(core_axis_name="c", subcore_axis_name="s")
    f = functools.partial(
        pl.kernel,
        mesh=mesh,
        out_type=jax.ShapeDtypeStruct((N_EDGES, D), jnp.float32),
        scratch_types=[
            pltpu.VMEM((CH,), jnp.int32),
            pltpu.VMEM((CH,), jnp.int32),
            pltpu.VMEM((CH, D), jnp.float32),
            pltpu.VMEM((CH, D), jnp.float32),
            pltpu.VMEM_SHARED((N_NODES, D), jnp.float32),
            pltpu.SemaphoreType.DMA,
            pltpu.SemaphoreType.DMA,
            pltpu.SemaphoreType.DMA,
            pltpu.SemaphoreType.DMA,
        ],
    )(_sgather_body)
    return f(pa, src)


def _s2_body(y_hbm, dst_hbm, out_hbm, *rest):
    bufs = rest[0:NBUF1]
    idxs = rest[NBUF1:2 * NBUF1]
    sh_h = rest[2 * NBUF1]
    sem_is = rest[2 * NBUF1 + 1:2 * NBUF1 + 1 + NBUF1]
    sem_ds = rest[2 * NBUF1 + 1 + NBUF1:2 * NBUF1 + 1 + 2 * NBUF1]
    sem_ss = rest[2 * NBUF1 + 1 + 2 * NBUF1:2 * NBUF1 + 1 + 3 * NBUF1]

    cid = lax.axis_index("c")
    sid = lax.axis_index("s")
    epw = N_EDGES // (NC * NS)
    nchunk = epw // CH
    ebase = (cid * NS + sid) * epw

    _zero_rows(bufs[0], CH)
    r0 = sid * STRIPE
    _zero_spmem_stripe(sh_h, r0, STRIPE, bufs[0], CH)

    @pl.when(sid == NS - 1)
    def _():
        pltpu.sync_copy(bufs[0].at[pl.ds(0, TAILN)],
                        sh_h.at[pl.ds(TAIL0, TAILN)])

    plsc.subcore_barrier()

    def issue(j, b):
        off = ebase + j * CH
        pltpu.async_copy(dst_hbm.at[pl.ds(off, CH)], idxs[b], sem_is[b])
        pltpu.async_copy(y_hbm.at[pl.ds(off, CH)], bufs[b], sem_ds[b])

    for b in range(LA1):
        issue(b, b)

    def chunk_step(j, b):
        bb = (b + LA1) % NBUF1

        @pl.when(j + LA1 < nchunk)
        def _():
            @pl.when(j >= NBUF1 - LA1)
            def _():
                pltpu.make_async_copy(bufs[bb], sh_h.at[idxs[bb]],
                                      sem_ss[bb]).wait()
            issue(j + LA1, bb)

        pltpu.make_async_copy(dst_hbm.at[pl.ds(0, CH)],
                              idxs[b], sem_is[b]).wait()
        pltpu.make_async_copy(y_hbm.at[pl.ds(0, CH)],
                              bufs[b], sem_ds[b]).wait()
        pltpu.async_copy(bufs[b], sh_h.at[idxs[b]], sem_ss[b], add=True)

    def round_body(g, _):
        for b in range(NBUF1):
            chunk_step(g + b, b)
        return 0

    lax.fori_loop(0, nchunk // NBUF1,
                  lambda g, x: round_body(g * NBUF1, x), 0, unroll=False)
    for k in range(nchunk % NBUF1):
        j = (nchunk // NBUF1) * NBUF1 + k
        chunk_step(j, j % NBUF1)
    for b in range(NBUF1):
        pltpu.make_async_copy(bufs[b], sh_h.at[idxs[b]], sem_ss[b]).wait()

    plsc.subcore_barrier()
    pltpu.sync_copy(sh_h.at[pl.ds(r0, STRIPE)],
                    out_hbm.at[cid, pl.ds(r0, STRIPE)])

    @pl.when(sid == NS - 1)
    def _():
        pltpu.sync_copy(sh_h.at[pl.ds(TAIL0, TAILN)],
                        out_hbm.at[cid, pl.ds(TAIL0, TAILN)])


def _s2(y, dst):
    mesh = plsc.VectorSubcoreMesh(core_axis_name="c", subcore_axis_name="s")
    f = functools.partial(
        pl.kernel,
        mesh=mesh,
        out_type=jax.ShapeDtypeStruct((NC, N_NODES, D), jnp.float32),
        scratch_types=(
            [pltpu.VMEM((CH, D), jnp.float32)] * NBUF1
            + [pltpu.VMEM((CH,), jnp.int32)] * NBUF1
            + [pltpu.VMEM_SHARED((N_NODES, D), jnp.float32)]
            + [pltpu.SemaphoreType.DMA] * (3 * NBUF1)
        ),
    )(_s2_body)
    return f(y, dst)


NBLK = 2000


def _tadd_body(a_ref, b_ref, o_ref):
    o_ref[...] = a_ref[0] + b_ref[0]


def _tadd(parts):
    grid = (N_NODES // NBLK,)
    return pl.pallas_call(
        _tadd_body,
        grid=grid,
        in_specs=[
            pl.BlockSpec((1, NBLK, D), lambda i: (0, i, 0)),
            pl.BlockSpec((1, NBLK, D), lambda i: (1, i, 0)),
        ],
        out_specs=pl.BlockSpec((NBLK, D), lambda i: (i, 0)),
        out_shape=jax.ShapeDtypeStruct((N_NODES, D), jnp.float32),
    )(parts, parts)


def _t2_body(m_ref, w_ref, r_ref, e0_ref, y_ref):
    acc = jnp.dot(m_ref[...], w_ref[...], preferred_element_type=jnp.float32,
                  precision=jax.lax.Precision.HIGHEST)
    y_ref[...] = jnp.maximum(acc + r_ref[...] + e0_ref[...], 0.0)


def _t2(m, wbp_t, r, e0):
    grid = (N_EDGES // BLK,)
    return pl.pallas_call(
        _t2_body,
        grid=grid,
        in_specs=[
            pl.BlockSpec((BLK, D), lambda i: (i, 0)),
            pl.BlockSpec((D, D), lambda i: (0, 0)),
            pl.BlockSpec((BLK, D), lambda i: (i, 0)),
            pl.BlockSpec((1, D), lambda i: (0, 0)),
        ],
        out_specs=pl.BlockSpec((BLK, D), lambda i: (i, 0)),
        out_shape=jax.ShapeDtypeStruct((N_EDGES, D), jnp.float32),
    )(m, wbp_t, r, e0)


def _tg_body(m_ref, g_ref, s_ref):
    @pl.when(pl.program_id(0) == 0)
    def _():
        g_ref[...] = jnp.zeros_like(g_ref)
        s_ref[...] = jnp.zeros_like(s_ref)

    blk = m_ref[...]
    g_ref[...] += jnp.dot(blk.T, blk, preferred_element_type=jnp.float32,
                          precision=jax.lax.Precision.HIGHEST)
    s_ref[...] += jnp.sum(blk, axis=0, keepdims=True)


def _tg(m):
    grid = (N_EDGES // BLK,)
    return pl.pallas_call(
        _tg_body,
        grid=grid,
        in_specs=[pl.BlockSpec((BLK, D), lambda i: (i, 0))],
        out_specs=[
            pl.BlockSpec((D, D), lambda i: (0, 0)),
            pl.BlockSpec((1, D), lambda i: (0, 0)),
        ],
        out_shape=[
            jax.ShapeDtypeStruct((D, D), jnp.float32),
            jax.ShapeDtypeStruct((1, D), jnp.float32),
        ],
    )(m)


NBLK_S = 2000
NB_S = N_NODES // NBLK_S
E_F = float(N_EDGES)


def _tstats_body(hd_ref, hsrc_ref, c0_ref, c1_ref, g_ref, sm_ref, w_ref,
                 b1_ref, g1_ref, be1_ref,
                 pa_ref, e0_ref, wbp_ref,
                 sr_acc, srr_acc, x_acc, a_scr, e0_scr):
    i = pl.program_id(0)
    hp = jax.lax.Precision.HIGHEST
    wa_t = w_ref[:, :D].T
    wb = w_ref[:, D:]

    @pl.when(i == 0)
    def _():
        sr_acc[...] = jnp.zeros_like(sr_acc)
        srr_acc[...] = jnp.zeros_like(srr_acc)
        x_acc[...] = jnp.zeros_like(x_acc)

    @pl.when(i < NB_S)
    def _():
        p_blk = jnp.dot(hd_ref[...], wa_t, preferred_element_type=jnp.float32,
                        precision=hp)
        q_blk = jnp.dot(hsrc_ref[...], wb.T,
                        preferred_element_type=jnp.float32, precision=hp)
        c_blk = c0_ref[0][:, 0:1] + c1_ref[0][:, 0:1]
        sr_acc[...] += jnp.sum(c_blk * p_blk, axis=0, keepdims=True)
        srr_acc[...] += jnp.sum(c_blk * p_blk * p_blk, axis=0, keepdims=True)
        x_acc[...] += jnp.sum(p_blk * q_blk, axis=0, keepdims=True)
        pa_ref[...] = jnp.zeros_like(pa_ref)
        e0_ref[...] = jnp.zeros_like(e0_ref)
        wbp_ref[...] = jnp.zeros_like(wbp_ref)

    @pl.when(i == NB_S)
    def _():
        b1 = b1_ref[...]
        colsum_q = jnp.dot(sm_ref[...], wb.T,
                           preferred_element_type=jnp.float32, precision=hp)
        wbg = jnp.dot(wb, g_ref[...], preferred_element_type=jnp.float32,
                      precision=hp)
        colsumsq_q = jnp.sum(wbg * wb, axis=1, keepdims=True).T
        sum_z = colsum_q + sr_acc[...] + E_F * b1
        sumsq_z = (colsumsq_q + srr_acc[...] + E_F * b1 * b1
                   + 2.0 * x_acc[...] + 2.0 * b1 * colsum_q
                   + 2.0 * b1 * sr_acc[...])
        mean = sum_z / E_F
        var = sumsq_z / E_F - mean * mean
        a = g1_ref[...] / jnp.sqrt(var + 1e-5)
        a_scr[...] = a
        e0_scr[...] = a * (b1 - mean) + be1_ref[...]

    @pl.when(i >= NB_S)
    def _():
        a = a_scr[...]
        p_blk = jnp.dot(hd_ref[...], wa_t, preferred_element_type=jnp.float32,
                        precision=hp)
        pa_ref[...] = p_blk * a
        e0_ref[...] = e0_scr[...]
        wbp_ref[...] = wb.T * a


def _tstats(h_dst, h_src, cnt, G, sm, W1, b1, g1, be1):
    grid = (2 * NB_S,)
    bm = lambda i: (i % NB_S, 0)
    zero2 = lambda i: (0, 0)
    return pl.pallas_call(
        _tstats_body,
        grid=grid,
        in_specs=[
            pl.BlockSpec((NBLK_S, D), bm),
            pl.BlockSpec((NBLK_S, D), bm),
            pl.BlockSpec((1, NBLK_S, D), lambda i: (0, i % NB_S, 0)),
            pl.BlockSpec((1, NBLK_S, D), lambda i: (1, i % NB_S, 0)),
            pl.BlockSpec((D, D), zero2),
            pl.BlockSpec((1, D), zero2),
            pl.BlockSpec((D, 2 * D), zero2),
            pl.BlockSpec((1, D), zero2),
            pl.BlockSpec((1, D), zero2),
            pl.BlockSpec((1, D), zero2),
        ],
        out_specs=[
            pl.BlockSpec((NBLK_S, D), bm),
            pl.BlockSpec((1, D), zero2),
            pl.BlockSpec((D, D), zero2),
        ],
        out_shape=[
            jax.ShapeDtypeStruct((N_NODES, D), jnp.float32),
            jax.ShapeDtypeStruct((1, D), jnp.float32),
            jax.ShapeDtypeStruct((D, D), jnp.float32),
        ],
        scratch_shapes=[
            pltpu.VMEM((1, D), jnp.float32),
            pltpu.VMEM((1, D), jnp.float32),
            pltpu.VMEM((1, D), jnp.float32),
            pltpu.VMEM((1, D), jnp.float32),
            pltpu.VMEM((1, D), jnp.float32),
        ],
    )(h_dst, h_src, cnt, cnt, G, sm, W1,
      b1[None, :], g1[None, :], be1[None, :])


def kernel(m, edge_index, W1, b1, g1, be1, W2, b2, g2, be2):
    ei = edge_index.astype(jnp.int32)
    src = ei[0]
    dst = ei[1]

    hh = _s1(m, ei)
    cnt = _scnt(src)
    G, sm = _tg(m)
    pa, e0, wbp_t = _tstats(hh[0], hh[1], cnt, G, sm, W1, b1, g1, be1)
    r = _sgather(pa, src)
    y = _t2(m, wbp_t, r, e0)
    parts = _s2(y, dst)
    h2 = _tadd(parts)
    return h2
